# baseline (speedup 1.0000x reference)
import functools

import jax
import jax.numpy as jnp
from jax import lax
from jax.experimental import pallas as pl
from jax.experimental.pallas import tpu as pltpu
from jax.experimental.pallas import tpu_sc as plsc

_N = 50000
_E = 800000
_H = 64
_G = 128
_NC = 2
_NS = 16
_NW = _NC * _NS
_EPAD = 802816
_CPW = _EPAD // _NW
_NJ = _CPW // 128
_CPT = _EPAD // _NS
_NJ5 = _CPT // 128
_NH = _N // 2
_CSH = 25600
_TRASH = 25300
_NB = 400
_NGB = _N // _NB
_EB = 512
_EGB = _EPAD // _EB

_F32 = jnp.float32


@functools.lru_cache(maxsize=1)
def _mesh():
    return plsc.VectorSubcoreMesh(core_axis_name="c", subcore_axis_name="s",
                                  num_cores=_NC, num_subcores=_NS)


def _leaky(x):
    return jnp.where(x >= 0, x, 0.01 * x)


def _elu(x):
    return jnp.where(x > 0, x, jnp.exp(x) - 1.0)


def _full(shape):
    return pl.BlockSpec(shape, lambda i: (0,) * len(shape))


def _tc_pre0(h, wpn, bpn, w1a, w2a):
    def body(h_ref, wpn_ref, bpn_ref, w1a_ref, w2a_ref, hv_ref, hs_ref, ld_ref):
        hb = h_ref[...]
        hv = _leaky(jnp.dot(hb, wpn_ref[...], preferred_element_type=_F32)
                    + bpn_ref[...])
        hv_ref[...] = hv
        hs_ref[...] = jnp.dot(hb, w1a_ref[...], preferred_element_type=_F32)
        ld_ref[...] = jnp.dot(hv, w2a_ref[...], preferred_element_type=_F32)

    return pl.pallas_call(
        body,
        grid=(_NGB,),
        in_specs=[pl.BlockSpec((_NB, 133), lambda i: (i, 0)),
                  _full((133, 64)), _full((1, 64)), _full((133, 64)),
                  _full((64, 1))],
        out_specs=[pl.BlockSpec((_NB, 64), lambda i: (i, 0)),
                   pl.BlockSpec((_NB, 64), lambda i: (i, 0)),
                   pl.BlockSpec((_NB, 1), lambda i: (i, 0))],
        out_shape=[jax.ShapeDtypeStruct((_N, 64), _F32),
                   jax.ShapeDtypeStruct((_N, 64), _F32),
                   jax.ShapeDtypeStruct((_N, 1), _F32)],
    )(h, wpn, bpn, w1a, w2a)


def _tc_edge0(hs_src, e_p, ld_dst, w1b, b1, wt, bt, w2b, b2):
    def body(hs_ref, e_ref, ld_ref, w1b_ref, b1_ref, wt_ref, bt_ref,
             w2b_ref, b2_ref, t_ref, lg_ref, gm_ref):
        i = pl.program_id(0)
        he1 = _leaky(hs_ref[...]
                     + jnp.dot(e_ref[...], w1b_ref[...],
                               preferred_element_type=_F32) + b1_ref[...])
        t_ref[...] = jnp.dot(he1, wt_ref[...],
                             preferred_element_type=_F32) + bt_ref[...]
        lg = _leaky(ld_ref[...]
                    + jnp.dot(he1, w2b_ref[...], preferred_element_type=_F32)
                    + b2_ref[...])
        lg_ref[...] = lg

        @pl.when(i == 0)
        def _():
            gm_ref[...] = jnp.full((1, 1), -3e38, _F32)

        gm_ref[...] = jnp.maximum(gm_ref[...], jnp.max(lg)[None, None])

    return pl.pallas_call(
        body,
        grid=(_EGB,),
        in_specs=[pl.BlockSpec((_EB, 64), lambda i: (i, 0)),
                  pl.BlockSpec((_EB, 14), lambda i: (i, 0)),
                  pl.BlockSpec((_EB, 1), lambda i: (i, 0)),
                  _full((14, 64)), _full((1, 64)), _full((64, 64)),
                  _full((1, 64)), _full((64, 1)), _full((1, 1))],
        out_specs=[pl.BlockSpec((_EB, 64), lambda i: (i, 0)),
                   pl.BlockSpec((_EB, 1), lambda i: (i, 0)),
                   _full((1, 1))],
        out_shape=[jax.ShapeDtypeStruct((_EPAD, 64), _F32),
                   jax.ShapeDtypeStruct((_EPAD, 1), _F32),
                   jax.ShapeDtypeStruct((1, 1), _F32)],
    )(hs_src, e_p, ld_dst, w1b, b1, wt, bt, w2b, b2)


def _tc_inv_s(s_parts):
    def body(sp_ref, out_ref):
        s = jnp.sum(sp_ref[...], axis=0, keepdims=True)
        out_ref[...] = 1.0 / jnp.where(s == 0.0, 1.0, s)

    return pl.pallas_call(
        body,
        grid=(1,),
        in_specs=[_full((_NW, _N))],
        out_specs=_full((1, _N)),
        out_shape=jax.ShapeDtypeStruct((1, _N), _F32),
    )(s_parts)


def _tc_gru(cpa, cpb, inv_n1, hprev, gw, n_rows, blk):
    def body(xa_ref, xb_ref, inv_ref, h_ref, wir, wiz, win, whr, whz, whn,
             bir, biz, bin_, bhr, bhz, bhn, out_ref):
        x = _elu(jnp.concatenate([xa_ref[...], xb_ref[...]], axis=1)
                 * inv_ref[...])
        hp = h_ref[...]
        dot = lambda a, b: jnp.dot(a, b[...], preferred_element_type=_F32)
        r = jax.nn.sigmoid(dot(x, wir) + bir[...] + dot(hp, whr) + bhr[...])
        z = jax.nn.sigmoid(dot(x, wiz) + biz[...] + dot(hp, whz) + bhz[...])
        n = jnp.tanh(dot(x, win) + bin_[...] + r * (dot(hp, whn) + bhn[...]))
        out_ref[...] = jax.nn.relu((1.0 - z) * n + z * hp)

    mats = [gw[k] for k in ('wir', 'wiz', 'win', 'whr', 'whz', 'whn')]
    vecs = [gw[k] for k in ('bir', 'biz', 'bin', 'bhr', 'bhz', 'bhn')]
    return pl.pallas_call(
        body,
        grid=(n_rows // blk,),
        in_specs=[pl.BlockSpec((blk, 32), lambda i: (i, 0)),
                  pl.BlockSpec((blk, 32), lambda i: (i, 0)),
                  pl.BlockSpec((blk, 1), lambda i: (i, 0)),
                  pl.BlockSpec((blk, 64), lambda i: (i, 0))]
                 + [_full((64, 64))] * 6 + [_full((1, 64))] * 6,
        out_specs=pl.BlockSpec((blk, 64), lambda i: (i, 0)),
        out_shape=jax.ShapeDtypeStruct((n_rows, 64), _F32),
    )(cpa, cpb, inv_n1, hprev, *mats, *vecs)


def _tc_layer_pre(node, wp, bp, wdst, wsrc, be):
    def body(n_ref, wp_ref, bp_ref, wd_ref, ws_ref, be_ref,
             hp_ref, ld_ref, ls_ref, md_ref, ms_ref):
        i = pl.program_id(0)
        nb = n_ref[...]
        hp_ref[...] = jnp.dot(nb, wp_ref[...],
                              preferred_element_type=_F32) + bp_ref[...]
        ldb = jnp.dot(nb, wd_ref[...], preferred_element_type=_F32) + be_ref[...]
        lsb = jnp.dot(nb, ws_ref[...], preferred_element_type=_F32)
        ld_ref[...] = ldb
        ls_ref[...] = lsb

        @pl.when(i == 0)
        def _():
            md_ref[...] = jnp.full((1, 1), -3e38, _F32)
            ms_ref[...] = jnp.full((1, 1), -3e38, _F32)

        md_ref[...] = jnp.maximum(md_ref[...], jnp.max(ldb)[None, None])
        ms_ref[...] = jnp.maximum(ms_ref[...], jnp.max(lsb)[None, None])

    return pl.pallas_call(
        body,
        grid=(_NGB,),
        in_specs=[pl.BlockSpec((_NB, 64), lambda i: (i, 0)),
                  _full((64, 64)), _full((1, 64)), _full((64, 1)),
                  _full((64, 1)), _full((1, 1))],
        out_specs=[pl.BlockSpec((_NB, 64), lambda i: (i, 0)),
                   pl.BlockSpec((_NB, 1), lambda i: (i, 0)),
                   pl.BlockSpec((_NB, 1), lambda i: (i, 0)),
                   _full((1, 1)), _full((1, 1))],
        out_shape=[jax.ShapeDtypeStruct((_N, 64), _F32),
                   jax.ShapeDtypeStruct((_N, 1), _F32),
                   jax.ShapeDtypeStruct((_N, 1), _F32),
                   jax.ShapeDtypeStruct((1, 1), _F32),
                   jax.ShapeDtypeStruct((1, 1), _F32)],
    )(node, wp, bp, wdst, wsrc, be)


def _onehot(gid_blk):
    iota = lax.broadcasted_iota(jnp.int32, (1, _G), 1)
    return jnp.where(gid_blk == iota, 1.0, 0.0).astype(_F32)


def _tc_readout_sum(node, gids3):
    def body(n_ref, g_ref, out_ref):
        i = pl.program_id(0)

        @pl.when(i == 0)
        def _():
            out_ref[...] = jnp.zeros((_G, 64), _F32)

        oh = _onehot(g_ref[0])
        hf = jax.nn.relu(n_ref[...])
        out_ref[...] += lax.dot_general(oh, hf, (((0,), (0,)), ((), ())),
                                        preferred_element_type=_F32)

    return pl.pallas_call(
        body,
        grid=(_NGB,),
        in_specs=[pl.BlockSpec((_NB, 64), lambda i: (i, 0)),
                  pl.BlockSpec((1, _NB, 1), lambda i: (i, 0, 0))],
        out_specs=_full((_G, 64)),
        out_shape=jax.ShapeDtypeStruct((_G, 64), _F32),
    )(node, gids3)


def _tc_r2a(node, gids3, g_feats, wclg, wclh, bz):
    def body(n_ref, g_ref, gf_ref, wg_ref, wh_ref, bz_ref, z_ref, m_ref):
        i = pl.program_id(0)

        @pl.when(i == 0)
        def _():
            m_ref[...] = jnp.full((1, _G), -3e38, _F32)

        zg = jnp.dot(jax.nn.relu(gf_ref[...]), wg_ref[...],
                     preferred_element_type=_F32)
        oh = _onehot(g_ref[0])
        zgn = jnp.dot(oh, zg, preferred_element_type=_F32)
        zh = jnp.dot(jax.nn.relu(n_ref[...]), wh_ref[...],
                     preferred_element_type=_F32)
        z = _leaky(zgn + zh + bz_ref[...])
        z_ref[0] = z
        cand = jnp.where(oh > 0, z, -3e38)
        m_ref[...] = jnp.maximum(m_ref[...],
                                 jnp.max(cand, axis=0, keepdims=True))

    return pl.pallas_call(
        body,
        grid=(_NGB,),
        in_specs=[pl.BlockSpec((_NB, 64), lambda i: (i, 0)),
                  pl.BlockSpec((1, _NB, 1), lambda i: (i, 0, 0)),
                  _full((_G, 64)), _full((64, 1)), _full((64, 1)),
                  _full((1, 1))],
        out_specs=[pl.BlockSpec((1, _NB, 1), lambda i: (i, 0, 0)),
                   _full((1, _G))],
        out_shape=[jax.ShapeDtypeStruct((_NGB, _NB, 1), _F32),
                   jax.ShapeDtypeStruct((1, _G), _F32)],
    )(node, gids3, g_feats, wclg, wclh, bz)


def _tc_r2b(node, gids3, z3, m, wpn, bpn):
    def body(n_ref, g_ref, z_ref, m_ref, wp_ref, bp_ref, s_ref, u_ref):
        i = pl.program_id(0)

        @pl.when(i == 0)
        def _():
            s_ref[...] = jnp.zeros((1, _G), _F32)
            u_ref[...] = jnp.zeros((_G, 64), _F32)

        oh = _onehot(g_ref[0])
        mn = lax.dot_general(oh, m_ref[...], (((1,), (1,)), ((), ())),
                             preferred_element_type=_F32)
        ex = jnp.exp(z_ref[0] - mn)
        s_ref[...] += lax.dot_general(ex, oh, (((0,), (0,)), ((), ())),
                                      preferred_element_type=_F32)
        hv2 = jnp.dot(jax.nn.relu(n_ref[...]), wp_ref[...],
                      preferred_element_type=_F32) + bp_ref[...]
        u_ref[...] += lax.dot_general(oh, hv2 * ex, (((0,), (0,)), ((), ())),
                                      preferred_element_type=_F32)

    return pl.pallas_call(
        body,
        grid=(_NGB,),
        in_specs=[pl.BlockSpec((_NB, 64), lambda i: (i, 0)),
                  pl.BlockSpec((1, _NB, 1), lambda i: (i, 0, 0)),
                  pl.BlockSpec((1, _NB, 1), lambda i: (i, 0, 0)),
                  _full((1, _G)), _full((64, 64)), _full((1, 64))],
        out_specs=[_full((1, _G)), _full((_G, 64))],
        out_shape=[jax.ShapeDtypeStruct((1, _G), _F32),
                   jax.ShapeDtypeStruct((_G, 64), _F32)],
    )(node, gids3, z3, m, wpn, bpn)


def _tc_r2c(U, s, g_feats, gw):
    def body(u_ref, s_ref, gf_ref, wir, wiz, win, whr, whz, whn,
             bir, biz, bin_, bhr, bhz, bhn, out_ref):
        sd = jnp.where(s_ref[...] == 0.0, 1.0, s_ref[...])
        g_repr = u_ref[...] * (1.0 / jnp.transpose(sd))
        x = _elu(g_repr)
        hp = gf_ref[...]
        dot = lambda a, b: jnp.dot(a, b[...], preferred_element_type=_F32)
        r = jax.nn.sigmoid(dot(x, wir) + bir[...] + dot(hp, whr) + bhr[...])
        z = jax.nn.sigmoid(dot(x, wiz) + biz[...] + dot(hp, whz) + bhz[...])
        n = jnp.tanh(dot(x, win) + bin_[...] + r * (dot(hp, whn) + bhn[...]))
        out_ref[...] = (1.0 - z) * n + z * hp

    mats = [gw[k] for k in ('wir', 'wiz', 'win', 'whr', 'whz', 'whn')]
    vecs = [gw[k] for k in ('bir', 'biz', 'bin', 'bhr', 'bhz', 'bhn')]
    return pl.pallas_call(
        body,
        grid=(1,),
        in_specs=[_full((_G, 64)), _full((1, _G)), _full((_G, 64))]
                 + [_full((64, 64))] * 6 + [_full((1, 64))] * 6,
        out_specs=_full((_G, 64)),
        out_shape=jax.ShapeDtypeStruct((_G, 64), _F32),
    )(U, s, g_feats, *mats, *vecs)


def _sc_gather_rows(table, idx3):
    @functools.partial(
        pl.kernel, mesh=_mesh(),
        compiler_params=pltpu.CompilerParams(needs_layout_passes=False, use_tc_tiling_on_sc=False),
        out_type=jax.ShapeDtypeStruct((_EPAD, 64), _F32),
        scratch_types=[pltpu.VMEM((_NJ, 128), jnp.int32),
                       pltpu.VMEM((7, 128, 64), _F32),
                       pltpu.SemaphoreType.DMA,
                       pltpu.SemaphoreType.DMA],
    )
    def k(table_h, idx_h, out_h, idx_v, rows_v, sem_g, sem_o):
        wid = lax.axis_index("s") * _NC + lax.axis_index("c")
        pltpu.sync_copy(idx_h.at[wid], idx_v)
        base = wid * _CPW
        nburst = 7

        def group(g, carry):
            j0 = g * nburst
            gh = [pltpu.async_copy(table_h.at[idx_v.at[j0 + b]],
                                   rows_v.at[b], sem_g)
                  for b in range(nburst)]
            oh = []
            for b in range(nburst):
                gh[b].wait()
                oh.append(pltpu.async_copy(
                    rows_v.at[b],
                    out_h.at[pl.ds((base + (j0 + b) * 128), 128)], sem_o))
            for b in range(nburst):
                oh[b].wait()
            return carry

        lax.fori_loop(0, _NJ // nburst, group, 0)

    return k(table, idx3)


def _sc_gather_scalar(table, idx3):
    @functools.partial(
        pl.kernel, mesh=_mesh(),
        compiler_params=pltpu.CompilerParams(needs_layout_passes=False, use_tc_tiling_on_sc=False),
        out_type=jax.ShapeDtypeStruct((_NW, _NJ, 128), _F32),
        scratch_types=[pltpu.VMEM((_N,), _F32),
                       pltpu.VMEM((_NJ, 128), jnp.int32),
                       pltpu.VMEM((_NJ, 128), _F32)],
    )
    def k(table_h, idx_h, out_h, tab_v, idx_v, out_v):
        wid = lax.axis_index("s") * _NC + lax.axis_index("c")
        pltpu.sync_copy(table_h, tab_v)
        pltpu.sync_copy(idx_h.at[wid], idx_v)

        def body_j(j, carry):
            def body_v(v, carry2):
                dvec = idx_v[j, pl.ds(v * 16, 16)]
                out_v[j, pl.ds(v * 16, 16)] = plsc.load_gather(tab_v, [dvec])
                return carry2
            return lax.fori_loop(0, 8, body_v, carry)

        lax.fori_loop(0, _NJ, body_j, 0)
        pltpu.sync_copy(out_v, out_h.at[wid])

    return k(table, idx3)


def _sc_logits_denom(ldst, lsrc, dst3, src3, valid3, m16):
    st = 7

    @functools.partial(
        pl.kernel, mesh=_mesh(),
        compiler_params=pltpu.CompilerParams(needs_layout_passes=False, use_tc_tiling_on_sc=False),
        out_type=[jax.ShapeDtypeStruct((_NW, _NJ, 128), _F32),
                  jax.ShapeDtypeStruct((_NW, _N), _F32)],
        scratch_types=[pltpu.VMEM((_N,), _F32),
                       pltpu.VMEM((_N,), _F32),
                       pltpu.VMEM((_NH + 8,), _F32),
                       pltpu.VMEM((st, 128), jnp.int32),
                       pltpu.VMEM((st, 128), jnp.int32),
                       pltpu.VMEM((st, 128), _F32),
                       pltpu.VMEM((st, 128), _F32),
                       pltpu.VMEM((16,), _F32)],
    )
    def k(ld_h, ls_h, dst_h, src_h, val_h, m_h, ex_h, sp_h,
          ld_v, ls_v, sh_v, dst_v, src_v, val_v, ex_v, m_v):
        wid = lax.axis_index("s") * _NC + lax.axis_index("c")
        pltpu.sync_copy(ld_h, ld_v)
        pltpu.sync_copy(ls_h, ls_v)
        pltpu.sync_copy(m_h, m_v)
        mvec = m_v[...]
        zeros = jnp.zeros((16,), _F32)

        for p in (0, 1):
            lo = p * _NH

            def zbody(i, carry):
                sh_v[pl.ds(i * 16, 16)] = zeros
                return carry

            lax.fori_loop(0, (_NH + 8) // 16, zbody, 0)

            def outer(s, carry):
                pltpu.sync_copy(dst_h.at[wid, pl.ds(s * st, st)], dst_v)
                pltpu.sync_copy(src_h.at[wid, pl.ds(s * st, st)], src_v)
                if p == 0:
                    pltpu.sync_copy(val_h.at[wid, pl.ds(s * st, st)], val_v)

                def mid(j, c2):
                    for v in range(8):
                        sl = pl.ds(v * 16, 16)
                        dv = dst_v[j, sl]
                        lg = (plsc.load_gather(ld_v, [dv])
                              + plsc.load_gather(ls_v, [src_v[j, sl]]))
                        lg = jnp.where(lg >= 0, lg, 0.01 * lg)
                        ex = jnp.exp(lg - mvec)
                        if p == 0:
                            ex = ex * val_v[j, sl]
                            ex_v[j, sl] = ex
                        tgt = dv - lo
                        inr = jnp.logical_and(tgt >= 0, tgt < _NH)
                        plsc.addupdate_scatter(
                            sh_v, [jnp.where(inr, tgt, _NH)], ex, mask=inr)
                    return c2

                lax.fori_loop(0, st, mid, 0)
                if p == 0:
                    pltpu.sync_copy(ex_v, ex_h.at[wid, pl.ds(s * st, st)])
                return carry

            lax.fori_loop(0, _NJ // st, outer, 0)
            pltpu.sync_copy(sh_v.at[pl.ds(0, _NH)],
                            sp_h.at[wid, pl.ds(lo, _NH)])

    return k(ldst, lsrc, dst3, src3, valid3, m16)


def _sc_exp_denom(logits3, valid3, dst3, wmax):
    st = 14

    @functools.partial(
        pl.kernel, mesh=_mesh(),
        compiler_params=pltpu.CompilerParams(needs_layout_passes=False, use_tc_tiling_on_sc=False),
        out_type=[jax.ShapeDtypeStruct((_NW, _NJ, 128), _F32),
                  jax.ShapeDtypeStruct((_NW, _N), _F32)],
        scratch_types=[pltpu.VMEM((_N,), _F32),
                       pltpu.VMEM((st, 128), _F32),
                       pltpu.VMEM((st, 128), _F32),
                       pltpu.VMEM((st, 128), jnp.int32),
                       pltpu.VMEM((st, 128), _F32),
                       pltpu.VMEM((_NW, 16), _F32)],
    )
    def k(lg_h, val_h, dst_h, wm_h, ex_h, sp_h,
          s_v, lg_v, val_v, dst_v, ex_v, wm_v):
        wid = lax.axis_index("s") * _NC + lax.axis_index("c")
        pltpu.sync_copy(wm_h, wm_v)

        def redk(kk, m):
            return jnp.maximum(m, wm_v[kk])

        mx = lax.fori_loop(0, _NW, redk, jnp.full((16,), -3e38, _F32))
        gmax = lax.reduce_max(mx, axes=(0,))

        zeros = jnp.zeros((16,), _F32)

        def zbody(i, carry):
            s_v[pl.ds(i * 16, 16)] = zeros
            return carry

        lax.fori_loop(0, _N // 16, zbody, 0)

        def outer(s, carry):
            pltpu.sync_copy(lg_h.at[wid, pl.ds(s * st, st)], lg_v)
            pltpu.sync_copy(val_h.at[wid, pl.ds(s * st, st)], val_v)
            pltpu.sync_copy(dst_h.at[wid, pl.ds(s * st, st)], dst_v)

            def mid(j, c2):
                for v in range(8):
                    sl = pl.ds(v * 16, 16)
                    ex = jnp.exp(lg_v[j, sl] - gmax) * val_v[j, sl]
                    ex_v[j, sl] = ex
                    plsc.addupdate_scatter(s_v, [dst_v[j, sl]], ex)
                return c2

            lax.fori_loop(0, st, mid, 0)
            pltpu.sync_copy(ex_v, ex_h.at[wid, pl.ds(s * st, st)])
            return carry

        lax.fori_loop(0, _NJ // st, outer, 0)
        pltpu.sync_copy(s_v, sp_h.at[wid])

    return k(logits3, valid3, dst3, wmax)


def _sc_gather_scale_scatter(tab2, srcb, dstb, exb):
    _CS2 = 50048

    @functools.partial(
        pl.kernel, mesh=_mesh(),
        compiler_params=pltpu.CompilerParams(needs_layout_passes=False, use_tc_tiling_on_sc=False),
        out_type=jax.ShapeDtypeStruct((2, _N, 32), _F32),
        scratch_types=[pltpu.VMEM((7, 128), jnp.int32),
                       pltpu.VMEM((7, 128), jnp.int32),
                       pltpu.VMEM((7, 128), _F32),
                       pltpu.VMEM((2, 128, 32), _F32),
                       pltpu.VMEM((2, 128), jnp.int32),
                       pltpu.VMEM((128, 32), _F32),
                       pltpu.SemaphoreType.DMA,
                       pltpu.SemaphoreType.DMA,
                       pltpu.VMEM_SHARED((_CS2, 32), _F32)],
    )
    def k(tab_h, src_h, dst_h, ex_h, out_h, src_v, dst_v, ex_v,
          rows_v, idx2_v, zeros_v, sem_l, sem_s, c_sh):
        cid = lax.axis_index("c")
        tid = lax.axis_index("s")
        zeros = jnp.zeros((16,), _F32)

        def zb(i, carry):
            r = i // 2
            kk = i % 2
            zeros_v[r, pl.ds(kk * 16, 16)] = zeros
            return carry

        lax.fori_loop(0, 256, zb, 0)

        def zspmem(z, carry):
            pltpu.sync_copy(zeros_v,
                            c_sh.at[pl.ds(tid * 3128 + z * 128, 128)])
            return carry

        lax.fori_loop(0, 24, zspmem, 0)
        pltpu.sync_copy(zeros_v.at[pl.ds(0, 56)],
                        c_sh.at[pl.ds(tid * 3128 + 3072, 56)])
        plsc.subcore_barrier()

        def stage(s, carry):
            sl7 = pl.ds(s * 7, 7)
            pltpu.sync_copy(src_h.at[tid, sl7], src_v)
            pltpu.sync_copy(dst_h.at[tid, sl7], dst_v)
            pltpu.sync_copy(ex_h.at[tid, sl7], ex_v)
            def mkidx_gather(jj):
                b = jj % 2

                def mkidx(v, c2, jj=jj, b=b):
                    sl = pl.ds(v * 16, 16)
                    idx2_v[b, sl] = src_v[jj, sl] * 2 + cid
                    return c2

                lax.fori_loop(0, 8, mkidx, 0)
                return pltpu.async_copy(tab_h.at[idx2_v.at[b]],
                                        rows_v.at[b], sem_l)

            h_l = {0: mkidx_gather(0)}
            h_s = {}
            for jj in range(7):
                b = jj % 2
                if jj + 1 < 7:
                    if jj - 1 >= 0:
                        h_s[jj - 1].wait()
                    h_l[jj + 1] = mkidx_gather(jj + 1)
                h_l[jj].wait()

                def scale(g, c2, jj=jj, b=b):
                    for h in range(2):
                        exvec = ex_v[jj, pl.ds((2 * g + h) * 16, 16)]
                        for l in range(16):
                            a = exvec[l]
                            r = (2 * g + h) * 16 + l
                            rows_v[b, r, pl.ds(0, 16)] = (
                                rows_v[b, r, pl.ds(0, 16)] * a)
                            rows_v[b, r, pl.ds(16, 16)] = (
                                rows_v[b, r, pl.ds(16, 16)] * a)
                    return c2

                lax.fori_loop(0, 4, scale, 0)
                h_s[jj] = pltpu.async_copy(rows_v.at[b],
                                           c_sh.at[dst_v.at[jj]], sem_s,
                                           add=True)
            h_s[5].wait()
            h_s[6].wait()
            return carry

        lax.fori_loop(0, _NJ5 // 7, stage, 0)
        plsc.subcore_barrier()
        pltpu.sync_copy(c_sh.at[pl.ds(tid * 3125, 3125)],
                        out_h.at[cid, pl.ds(tid * 3125, 3125)])

    return k(tab2, srcb, dstb, exb)


def _prep_gru(gp):
    w_ih, w_hh = gp['w_ih'], gp['w_hh']
    b_ih, b_hh = gp['b_ih'], gp['b_hh']
    out = {}
    for i, nm in enumerate(('r', 'z', 'n')):
        out['wi' + nm] = jnp.transpose(w_ih[i * 64:(i + 1) * 64])
        out['wh' + nm] = jnp.transpose(w_hh[i * 64:(i + 1) * 64])
        out['bi' + nm] = b_ih[i * 64:(i + 1) * 64].reshape(1, 64)
        out['bh' + nm] = b_hh[i * 64:(i + 1) * 64].reshape(1, 64)
    return out


def kernel(h, e, edge_index, graph_ids, params):
    p = params
    src = edge_index[0]
    dst = edge_index[1]

    pad = _EPAD - _E
    src_p = jnp.pad(src, (0, pad))
    dst_p = jnp.pad(dst, (0, pad))
    e_p = jnp.pad(e, ((0, pad), (0, 0)))
    valid = jnp.pad(jnp.ones((_E,), _F32), (0, pad))
    src3 = src_p.reshape(_NW, _NJ, 128)
    dst3 = dst_p.reshape(_NW, _NJ, 128)
    srcb = src_p.reshape(_NS, _NJ5, 128)
    dstb = dst_p.reshape(_NS, _NJ5, 128)
    arangeb = jnp.arange(_EPAD, dtype=jnp.int32).reshape(_NS, _NJ5, 128)
    valid3 = valid.reshape(_NW, _NJ, 128)
    gids3 = graph_ids.reshape(_NGB, _NB, 1)

    w1 = p['proj_edge1']['w']
    w2 = p['proj_edge2']['w']
    wpn0 = p['proj_node']['w']
    bpn0 = p['proj_node']['b'].reshape(1, 64)
    w1a, w1b = w1[:133], w1[133:]
    b1 = p['proj_edge1']['b'].reshape(1, 64)
    w2a, w2b = w2[:64], w2[64:]
    b2 = p['proj_edge2']['b'].reshape(1, 1)
    wt = p['edge_transform']['w']
    bt = p['edge_transform']['b'].reshape(1, 64)
    gru0 = _prep_gru(p['gru0'])

    hv_new, hs, ld = _tc_pre0(h, wpn0, bpn0, w1a, w2a)
    hs_src = _sc_gather_rows(hs, src3)
    ld_dst3 = _sc_gather_scalar(ld.reshape(_N), dst3)
    t, logits2, gmax = _tc_edge0(hs_src, e_p, ld_dst3.reshape(_EPAD, 1),
                                 w1b, b1, wt, bt, w2b, b2)
    wmax = jnp.broadcast_to(gmax.reshape(1, 1), (_NW, 16))
    ex3, s_parts = _sc_exp_denom(logits2.reshape(_NW, _NJ, 128), valid3,
                                 dst3, wmax)
    inv_s = _tc_inv_s(s_parts)
    cp = _sc_gather_scale_scatter(t.reshape(2 * _EPAD, 32), arangeb, dstb,
                                  ex3.reshape(_NS, _NJ5, 128))
    node = _tc_gru(cp[0], cp[1], inv_s.reshape(_N, 1), hv_new, gru0,
                   _N, _NB)

    for lp in p['gnn']:
        wpe = lp['proj_edge']['w']
        hp, ldst, lsrc, mxd, mxs = _tc_layer_pre(
            node, lp['proj_node']['w'], lp['proj_node']['b'].reshape(1, 64),
            wpe[:64], wpe[64:], lp['proj_edge']['b'].reshape(1, 1))
        m16 = jnp.broadcast_to(_leaky(mxd + mxs).reshape(1), (16,))
        ex3, s_parts = _sc_logits_denom(ldst.reshape(_N), lsrc.reshape(_N),
                                        dst3, src3, valid3, m16)
        inv_s = _tc_inv_s(s_parts)
        cp = _sc_gather_scale_scatter(hp.reshape(2 * _N, 32), srcb, dstb,
                                      ex3.reshape(_NS, _NJ5, 128))
        node = _tc_gru(cp[0], cp[1], inv_s.reshape(_N, 1), node,
                       _prep_gru(lp['gru']), _N, _NB)

    g_feats = _tc_readout_sum(node, gids3)
    for rp in p['readout']:
        wcl = rp['compute_logits']['w']
        z3, m = _tc_r2a(node, gids3, g_feats, wcl[:64], wcl[64:],
                        rp['compute_logits']['b'].reshape(1, 1))
        s, U = _tc_r2b(node, gids3, z3, m, rp['project_nodes']['w'],
                       rp['project_nodes']['b'].reshape(1, 64))
        g_feats = _tc_r2c(U, s, g_feats, _prep_gru(rp['gru']))
    return g_feats


# speedup vs baseline: 1.1663x; 1.1663x over previous
import functools

import jax
import jax.numpy as jnp
from jax import lax
from jax.experimental import pallas as pl
from jax.experimental.pallas import tpu as pltpu
from jax.experimental.pallas import tpu_sc as plsc

_N = 50000
_E = 800000
_H = 64
_G = 128
_NC = 2
_NS = 16
_NW = _NC * _NS
_EPAD = 802816
_CPW = _EPAD // _NW
_NJ = _CPW // 128
_CPT = _EPAD // _NS
_NJ5 = _CPT // 128
_NH = _N // 2
_CSH = 25600
_TRASH = 25300
_NB = 400
_NGB = _N // _NB
_EB = 512
_EGB = _EPAD // _EB

_F32 = jnp.float32


@functools.lru_cache(maxsize=1)
def _mesh():
    return plsc.VectorSubcoreMesh(core_axis_name="c", subcore_axis_name="s",
                                  num_cores=_NC, num_subcores=_NS)


def _leaky(x):
    return jnp.where(x >= 0, x, 0.01 * x)


def _elu(x):
    return jnp.where(x > 0, x, jnp.exp(x) - 1.0)


def _full(shape):
    return pl.BlockSpec(shape, lambda i: (0,) * len(shape))


def _tc_pre0(h, wpn, bpn, w1a, w2a, b2):
    def body(h_ref, wpn_ref, bpn_ref, w1a_ref, w2a_ref, b2_ref,
             hv_ref, hs_ref, ld_ref, md_ref):
        i = pl.program_id(0)
        hb = h_ref[...]
        hv = _leaky(jnp.dot(hb, wpn_ref[...], preferred_element_type=_F32)
                    + bpn_ref[...])
        hv_ref[...] = hv
        hs_ref[...] = jnp.dot(hb, w1a_ref[...], preferred_element_type=_F32)
        ld = jnp.dot(hv, w2a_ref[...], preferred_element_type=_F32) + b2_ref[...]
        ld_ref[...] = ld

        @pl.when(i == 0)
        def _():
            md_ref[...] = jnp.full((1, 1), -3e38, _F32)

        md_ref[...] = jnp.maximum(md_ref[...], jnp.max(ld)[None, None])

    return pl.pallas_call(
        body,
        grid=(_NGB,),
        in_specs=[pl.BlockSpec((_NB, 133), lambda i: (i, 0)),
                  _full((133, 64)), _full((1, 64)), _full((133, 64)),
                  _full((64, 1)), _full((1, 1))],
        out_specs=[pl.BlockSpec((_NB, 64), lambda i: (i, 0)),
                   pl.BlockSpec((_NB, 64), lambda i: (i, 0)),
                   pl.BlockSpec((_NB, 1), lambda i: (i, 0)),
                   _full((1, 1))],
        out_shape=[jax.ShapeDtypeStruct((_N, 64), _F32),
                   jax.ShapeDtypeStruct((_N, 64), _F32),
                   jax.ShapeDtypeStruct((_N, 1), _F32),
                   jax.ShapeDtypeStruct((1, 1), _F32)],
    )(h, wpn, bpn, w1a, w2a, b2)


def _tc_edge0(hs_src, e_p, w1b, b1, wt, bt, w2b):
    def body(hs_ref, e_ref, w1b_ref, b1_ref, wt_ref, bt_ref,
             w2b_ref, t_ref, le_ref, gm_ref):
        i = pl.program_id(0)
        he1 = _leaky(hs_ref[...]
                     + jnp.dot(e_ref[...], w1b_ref[...],
                               preferred_element_type=_F32) + b1_ref[...])
        t_ref[...] = jnp.dot(he1, wt_ref[...],
                             preferred_element_type=_F32) + bt_ref[...]
        le = jnp.dot(he1, w2b_ref[...], preferred_element_type=_F32)
        le_ref[...] = le

        @pl.when(i == 0)
        def _():
            gm_ref[...] = jnp.full((1, 1), -3e38, _F32)

        gm_ref[...] = jnp.maximum(gm_ref[...], jnp.max(le)[None, None])

    return pl.pallas_call(
        body,
        grid=(_EGB,),
        in_specs=[pl.BlockSpec((_EB, 64), lambda i: (i, 0)),
                  pl.BlockSpec((_EB, 14), lambda i: (i, 0)),
                  _full((14, 64)), _full((1, 64)), _full((64, 64)),
                  _full((1, 64)), _full((64, 1))],
        out_specs=[pl.BlockSpec((_EB, 64), lambda i: (i, 0)),
                   pl.BlockSpec((_EB, 1), lambda i: (i, 0)),
                   _full((1, 1))],
        out_shape=[jax.ShapeDtypeStruct((_EPAD, 64), _F32),
                   jax.ShapeDtypeStruct((_EPAD, 1), _F32),
                   jax.ShapeDtypeStruct((1, 1), _F32)],
    )(hs_src, e_p, w1b, b1, wt, bt, w2b)


def _tc_inv_s(s_parts):
    def body(sp_ref, out_ref):
        s = jnp.sum(sp_ref[...], axis=0, keepdims=True)
        out_ref[...] = 1.0 / jnp.where(s == 0.0, 1.0, s)

    return pl.pallas_call(
        body,
        grid=(1,),
        in_specs=[_full((_NW, _N))],
        out_specs=_full((1, _N)),
        out_shape=jax.ShapeDtypeStruct((1, _N), _F32),
    )(s_parts)


def _tc_gru(cpa, cpb, inv_n1, hprev, gw, n_rows, blk):
    def body(xa_ref, xb_ref, inv_ref, h_ref, wir, wiz, win, whr, whz, whn,
             bir, biz, bin_, bhr, bhz, bhn, out_ref):
        x = _elu(jnp.concatenate([xa_ref[...], xb_ref[...]], axis=1)
                 * inv_ref[...])
        hp = h_ref[...]
        dot = lambda a, b: jnp.dot(a, b[...], preferred_element_type=_F32)
        r = jax.nn.sigmoid(dot(x, wir) + bir[...] + dot(hp, whr) + bhr[...])
        z = jax.nn.sigmoid(dot(x, wiz) + biz[...] + dot(hp, whz) + bhz[...])
        n = jnp.tanh(dot(x, win) + bin_[...] + r * (dot(hp, whn) + bhn[...]))
        out_ref[...] = jax.nn.relu((1.0 - z) * n + z * hp)

    mats = [gw[k] for k in ('wir', 'wiz', 'win', 'whr', 'whz', 'whn')]
    vecs = [gw[k] for k in ('bir', 'biz', 'bin', 'bhr', 'bhz', 'bhn')]
    return pl.pallas_call(
        body,
        grid=(n_rows // blk,),
        in_specs=[pl.BlockSpec((blk, 32), lambda i: (i, 0)),
                  pl.BlockSpec((blk, 32), lambda i: (i, 0)),
                  pl.BlockSpec((blk, 1), lambda i: (i, 0)),
                  pl.BlockSpec((blk, 64), lambda i: (i, 0))]
                 + [_full((64, 64))] * 6 + [_full((1, 64))] * 6,
        out_specs=pl.BlockSpec((blk, 64), lambda i: (i, 0)),
        out_shape=jax.ShapeDtypeStruct((n_rows, 64), _F32),
    )(cpa, cpb, inv_n1, hprev, *mats, *vecs)


def _tc_layer_pre(node, wp, bp, wdst, wsrc, be):
    def body(n_ref, wp_ref, bp_ref, wd_ref, ws_ref, be_ref,
             hp_ref, ld_ref, ls_ref, md_ref, ms_ref):
        i = pl.program_id(0)
        nb = n_ref[...]
        hp_ref[...] = jnp.dot(nb, wp_ref[...],
                              preferred_element_type=_F32) + bp_ref[...]
        ldb = jnp.dot(nb, wd_ref[...], preferred_element_type=_F32) + be_ref[...]
        lsb = jnp.dot(nb, ws_ref[...], preferred_element_type=_F32)
        ld_ref[...] = ldb
        ls_ref[...] = lsb

        @pl.when(i == 0)
        def _():
            md_ref[...] = jnp.full((1, 1), -3e38, _F32)
            ms_ref[...] = jnp.full((1, 1), -3e38, _F32)

        md_ref[...] = jnp.maximum(md_ref[...], jnp.max(ldb)[None, None])
        ms_ref[...] = jnp.maximum(ms_ref[...], jnp.max(lsb)[None, None])

    return pl.pallas_call(
        body,
        grid=(_NGB,),
        in_specs=[pl.BlockSpec((_NB, 64), lambda i: (i, 0)),
                  _full((64, 64)), _full((1, 64)), _full((64, 1)),
                  _full((64, 1)), _full((1, 1))],
        out_specs=[pl.BlockSpec((_NB, 64), lambda i: (i, 0)),
                   pl.BlockSpec((_NB, 1), lambda i: (i, 0)),
                   pl.BlockSpec((_NB, 1), lambda i: (i, 0)),
                   _full((1, 1)), _full((1, 1))],
        out_shape=[jax.ShapeDtypeStruct((_N, 64), _F32),
                   jax.ShapeDtypeStruct((_N, 1), _F32),
                   jax.ShapeDtypeStruct((_N, 1), _F32),
                   jax.ShapeDtypeStruct((1, 1), _F32),
                   jax.ShapeDtypeStruct((1, 1), _F32)],
    )(node, wp, bp, wdst, wsrc, be)


def _onehot(gid_blk):
    iota = lax.broadcasted_iota(jnp.int32, (1, _G), 1)
    return jnp.where(gid_blk == iota, 1.0, 0.0).astype(_F32)


def _tc_readout_sum(node, gids3):
    def body(n_ref, g_ref, out_ref):
        i = pl.program_id(0)

        @pl.when(i == 0)
        def _():
            out_ref[...] = jnp.zeros((_G, 64), _F32)

        oh = _onehot(g_ref[0])
        hf = jax.nn.relu(n_ref[...])
        out_ref[...] += lax.dot_general(oh, hf, (((0,), (0,)), ((), ())),
                                        preferred_element_type=_F32)

    return pl.pallas_call(
        body,
        grid=(_NGB,),
        in_specs=[pl.BlockSpec((_NB, 64), lambda i: (i, 0)),
                  pl.BlockSpec((1, _NB, 1), lambda i: (i, 0, 0))],
        out_specs=_full((_G, 64)),
        out_shape=jax.ShapeDtypeStruct((_G, 64), _F32),
    )(node, gids3)


def _tc_r2a(node, gids3, g_feats, wclg, wclh, bz):
    def body(n_ref, g_ref, gf_ref, wg_ref, wh_ref, bz_ref, z_ref, m_ref):
        i = pl.program_id(0)

        @pl.when(i == 0)
        def _():
            m_ref[...] = jnp.full((1, _G), -3e38, _F32)

        zg = jnp.dot(jax.nn.relu(gf_ref[...]), wg_ref[...],
                     preferred_element_type=_F32)
        oh = _onehot(g_ref[0])
        zgn = jnp.dot(oh, zg, preferred_element_type=_F32)
        zh = jnp.dot(jax.nn.relu(n_ref[...]), wh_ref[...],
                     preferred_element_type=_F32)
        z = _leaky(zgn + zh + bz_ref[...])
        z_ref[0] = z
        cand = jnp.where(oh > 0, z, -3e38)
        m_ref[...] = jnp.maximum(m_ref[...],
                                 jnp.max(cand, axis=0, keepdims=True))

    return pl.pallas_call(
        body,
        grid=(_NGB,),
        in_specs=[pl.BlockSpec((_NB, 64), lambda i: (i, 0)),
                  pl.BlockSpec((1, _NB, 1), lambda i: (i, 0, 0)),
                  _full((_G, 64)), _full((64, 1)), _full((64, 1)),
                  _full((1, 1))],
        out_specs=[pl.BlockSpec((1, _NB, 1), lambda i: (i, 0, 0)),
                   _full((1, _G))],
        out_shape=[jax.ShapeDtypeStruct((_NGB, _NB, 1), _F32),
                   jax.ShapeDtypeStruct((1, _G), _F32)],
    )(node, gids3, g_feats, wclg, wclh, bz)


def _tc_r2b(node, gids3, z3, m, wpn, bpn):
    def body(n_ref, g_ref, z_ref, m_ref, wp_ref, bp_ref, s_ref, u_ref):
        i = pl.program_id(0)

        @pl.when(i == 0)
        def _():
            s_ref[...] = jnp.zeros((1, _G), _F32)
            u_ref[...] = jnp.zeros((_G, 64), _F32)

        oh = _onehot(g_ref[0])
        mn = lax.dot_general(oh, m_ref[...], (((1,), (1,)), ((), ())),
                             preferred_element_type=_F32)
        ex = jnp.exp(z_ref[0] - mn)
        s_ref[...] += lax.dot_general(ex, oh, (((0,), (0,)), ((), ())),
                                      preferred_element_type=_F32)
        hv2 = jnp.dot(jax.nn.relu(n_ref[...]), wp_ref[...],
                      preferred_element_type=_F32) + bp_ref[...]
        u_ref[...] += lax.dot_general(oh, hv2 * ex, (((0,), (0,)), ((), ())),
                                      preferred_element_type=_F32)

    return pl.pallas_call(
        body,
        grid=(_NGB,),
        in_specs=[pl.BlockSpec((_NB, 64), lambda i: (i, 0)),
                  pl.BlockSpec((1, _NB, 1), lambda i: (i, 0, 0)),
                  pl.BlockSpec((1, _NB, 1), lambda i: (i, 0, 0)),
                  _full((1, _G)), _full((64, 64)), _full((1, 64))],
        out_specs=[_full((1, _G)), _full((_G, 64))],
        out_shape=[jax.ShapeDtypeStruct((1, _G), _F32),
                   jax.ShapeDtypeStruct((_G, 64), _F32)],
    )(node, gids3, z3, m, wpn, bpn)


def _tc_r2c(U, s, g_feats, gw):
    def body(u_ref, s_ref, gf_ref, wir, wiz, win, whr, whz, whn,
             bir, biz, bin_, bhr, bhz, bhn, out_ref):
        sd = jnp.where(s_ref[...] == 0.0, 1.0, s_ref[...])
        g_repr = u_ref[...] * (1.0 / jnp.transpose(sd))
        x = _elu(g_repr)
        hp = gf_ref[...]
        dot = lambda a, b: jnp.dot(a, b[...], preferred_element_type=_F32)
        r = jax.nn.sigmoid(dot(x, wir) + bir[...] + dot(hp, whr) + bhr[...])
        z = jax.nn.sigmoid(dot(x, wiz) + biz[...] + dot(hp, whz) + bhz[...])
        n = jnp.tanh(dot(x, win) + bin_[...] + r * (dot(hp, whn) + bhn[...]))
        out_ref[...] = (1.0 - z) * n + z * hp

    mats = [gw[k] for k in ('wir', 'wiz', 'win', 'whr', 'whz', 'whn')]
    vecs = [gw[k] for k in ('bir', 'biz', 'bin', 'bhr', 'bhz', 'bhn')]
    return pl.pallas_call(
        body,
        grid=(1,),
        in_specs=[_full((_G, 64)), _full((1, _G)), _full((_G, 64))]
                 + [_full((64, 64))] * 6 + [_full((1, 64))] * 6,
        out_specs=_full((_G, 64)),
        out_shape=jax.ShapeDtypeStruct((_G, 64), _F32),
    )(U, s, g_feats, *mats, *vecs)


def _sc_gather_rows(table, idx3):
    @functools.partial(
        pl.kernel, mesh=_mesh(),
        compiler_params=pltpu.CompilerParams(needs_layout_passes=False, use_tc_tiling_on_sc=False),
        out_type=jax.ShapeDtypeStruct((_EPAD, 64), _F32),
        scratch_types=[pltpu.VMEM((_NJ, 128), jnp.int32),
                       pltpu.VMEM((7, 128, 64), _F32),
                       pltpu.SemaphoreType.DMA,
                       pltpu.SemaphoreType.DMA],
    )
    def k(table_h, idx_h, out_h, idx_v, rows_v, sem_g, sem_o):
        wid = lax.axis_index("s") * _NC + lax.axis_index("c")
        pltpu.sync_copy(idx_h.at[wid], idx_v)
        base = wid * _CPW
        nburst = 7

        def group(g, carry):
            j0 = g * nburst
            gh = [pltpu.async_copy(table_h.at[idx_v.at[j0 + b]],
                                   rows_v.at[b], sem_g)
                  for b in range(nburst)]
            oh = []
            for b in range(nburst):
                gh[b].wait()
                oh.append(pltpu.async_copy(
                    rows_v.at[b],
                    out_h.at[pl.ds((base + (j0 + b) * 128), 128)], sem_o))
            for b in range(nburst):
                oh[b].wait()
            return carry

        lax.fori_loop(0, _NJ // nburst, group, 0)

    return k(table, idx3)


def _sc_logits_denom(ldst, lsrc, dst3, src3, valid3, m16):
    st = 7

    @functools.partial(
        pl.kernel, mesh=_mesh(),
        compiler_params=pltpu.CompilerParams(needs_layout_passes=False, use_tc_tiling_on_sc=False),
        out_type=[jax.ShapeDtypeStruct((_NW, _NJ, 128), _F32),
                  jax.ShapeDtypeStruct((_NW, _N), _F32)],
        scratch_types=[pltpu.VMEM((_N,), _F32),
                       pltpu.VMEM((_N,), _F32),
                       pltpu.VMEM((_NH + 8,), _F32),
                       pltpu.VMEM((st, 128), jnp.int32),
                       pltpu.VMEM((st, 128), jnp.int32),
                       pltpu.VMEM((st, 128), _F32),
                       pltpu.VMEM((st, 128), _F32),
                       pltpu.VMEM((16,), _F32)],
    )
    def k(ld_h, ls_h, dst_h, src_h, val_h, m_h, ex_h, sp_h,
          ld_v, ls_v, sh_v, dst_v, src_v, val_v, ex_v, m_v):
        wid = lax.axis_index("s") * _NC + lax.axis_index("c")
        pltpu.sync_copy(ld_h, ld_v)
        pltpu.sync_copy(ls_h, ls_v)
        pltpu.sync_copy(m_h, m_v)
        mvec = m_v[...]
        zeros = jnp.zeros((16,), _F32)

        for p in (0, 1):
            lo = p * _NH

            def zbody(i, carry):
                sh_v[pl.ds(i * 16, 16)] = zeros
                return carry

            lax.fori_loop(0, (_NH + 8) // 16, zbody, 0)

            def outer(s, carry):
                pltpu.sync_copy(dst_h.at[wid, pl.ds(s * st, st)], dst_v)
                pltpu.sync_copy(src_h.at[wid, pl.ds(s * st, st)], src_v)
                if p == 0:
                    pltpu.sync_copy(val_h.at[wid, pl.ds(s * st, st)], val_v)

                def mid(j, c2):
                    def inner(v, c3):
                        sl = pl.ds(v * 16, 16)
                        dv = dst_v[j, sl]
                        lg = (plsc.load_gather(ld_v, [dv])
                              + plsc.load_gather(ls_v, [src_v[j, sl]]))
                        lg = jnp.where(lg >= 0, lg, 0.01 * lg)
                        ex = jnp.exp(lg - mvec)
                        if p == 0:
                            ex = ex * val_v[j, sl]
                            ex_v[j, sl] = ex
                        tgt = dv - lo
                        inr = jnp.logical_and(tgt >= 0, tgt < _NH)
                        plsc.addupdate_scatter(
                            sh_v, [jnp.where(inr, tgt, _NH)], ex, mask=inr)
                        return c3
                    return lax.fori_loop(0, 8, inner, c2)

                lax.fori_loop(0, st, mid, 0)
                if p == 0:
                    pltpu.sync_copy(ex_v, ex_h.at[wid, pl.ds(s * st, st)])
                return carry

            lax.fori_loop(0, _NJ // st, outer, 0)
            pltpu.sync_copy(sh_v.at[pl.ds(0, _NH)],
                            sp_h.at[wid, pl.ds(lo, _NH)])

    return k(ldst, lsrc, dst3, src3, valid3, m16)


def _sc_le_denom(le3, valid3, dst3, ldtab, m16):
    st = 7

    @functools.partial(
        pl.kernel, mesh=_mesh(),
        compiler_params=pltpu.CompilerParams(needs_layout_passes=False, use_tc_tiling_on_sc=False),
        out_type=[jax.ShapeDtypeStruct((_NW, _NJ, 128), _F32),
                  jax.ShapeDtypeStruct((_NW, _N), _F32)],
        scratch_types=[pltpu.VMEM((_N,), _F32),
                       pltpu.VMEM((_N,), _F32),
                       pltpu.VMEM((st, 128), _F32),
                       pltpu.VMEM((st, 128), _F32),
                       pltpu.VMEM((st, 128), jnp.int32),
                       pltpu.VMEM((st, 128), _F32),
                       pltpu.VMEM((16,), _F32)],
    )
    def k(le_h, val_h, dst_h, ld_h, m_h, ex_h, sp_h,
          ld_v, s_v, le_v, val_v, dst_v, ex_v, m_v):
        wid = lax.axis_index("s") * _NC + lax.axis_index("c")
        pltpu.sync_copy(ld_h, ld_v)
        pltpu.sync_copy(m_h, m_v)
        mvec = m_v[...]
        zeros = jnp.zeros((16,), _F32)

        def zbody(i, carry):
            s_v[pl.ds(i * 16, 16)] = zeros
            return carry

        lax.fori_loop(0, _N // 16, zbody, 0)

        def outer(s, carry):
            sl7 = pl.ds(s * st, st)
            pltpu.sync_copy(le_h.at[wid, sl7], le_v)
            pltpu.sync_copy(val_h.at[wid, sl7], val_v)
            pltpu.sync_copy(dst_h.at[wid, sl7], dst_v)

            def mid(j, c2):
                def inner(v, c3):
                    sl = pl.ds(v * 16, 16)
                    dv = dst_v[j, sl]
                    lg = plsc.load_gather(ld_v, [dv]) + le_v[j, sl]
                    lg = jnp.where(lg >= 0, lg, 0.01 * lg)
                    ex = jnp.exp(lg - mvec) * val_v[j, sl]
                    ex_v[j, sl] = ex
                    plsc.addupdate_scatter(s_v, [dv], ex)
                    return c3
                return lax.fori_loop(0, 8, inner, c2)

            lax.fori_loop(0, st, mid, 0)
            pltpu.sync_copy(ex_v, ex_h.at[wid, sl7])
            return carry

        lax.fori_loop(0, _NJ // st, outer, 0)
        pltpu.sync_copy(s_v, sp_h.at[wid])

    return k(le3, valid3, dst3, ldtab, m16)


def _sc_gather_scale_scatter(tab2, srcb, dstb, exb):
    _CS2 = 50048

    @functools.partial(
        pl.kernel, mesh=_mesh(),
        compiler_params=pltpu.CompilerParams(needs_layout_passes=False, use_tc_tiling_on_sc=False),
        out_type=jax.ShapeDtypeStruct((2, _N, 32), _F32),
        scratch_types=[pltpu.VMEM((7, 128), jnp.int32),
                       pltpu.VMEM((7, 128), jnp.int32),
                       pltpu.VMEM((7, 128), _F32),
                       pltpu.VMEM((2, 128, 32), _F32),
                       pltpu.VMEM((2, 128), jnp.int32),
                       pltpu.VMEM((128, 32), _F32),
                       pltpu.SemaphoreType.DMA,
                       pltpu.SemaphoreType.DMA,
                       pltpu.VMEM_SHARED((_CS2, 32), _F32)],
    )
    def k(tab_h, src_h, dst_h, ex_h, out_h, src_v, dst_v, ex_v,
          rows_v, idx2_v, zeros_v, sem_l, sem_s, c_sh):
        cid = lax.axis_index("c")
        tid = lax.axis_index("s")
        zeros = jnp.zeros((16,), _F32)

        def zb(i, carry):
            r = i // 2
            kk = i % 2
            zeros_v[r, pl.ds(kk * 16, 16)] = zeros
            return carry

        lax.fori_loop(0, 256, zb, 0)

        def zspmem(z, carry):
            pltpu.sync_copy(zeros_v,
                            c_sh.at[pl.ds(tid * 3128 + z * 128, 128)])
            return carry

        lax.fori_loop(0, 24, zspmem, 0)
        pltpu.sync_copy(zeros_v.at[pl.ds(0, 56)],
                        c_sh.at[pl.ds(tid * 3128 + 3072, 56)])
        plsc.subcore_barrier()

        def stage(s, carry):
            sl7 = pl.ds(s * 7, 7)
            pltpu.sync_copy(src_h.at[tid, sl7], src_v)
            pltpu.sync_copy(dst_h.at[tid, sl7], dst_v)
            pltpu.sync_copy(ex_h.at[tid, sl7], ex_v)
            def mkidx_gather(jj):
                b = jj % 2

                def mkidx(v, c2, jj=jj, b=b):
                    sl = pl.ds(v * 16, 16)
                    idx2_v[b, sl] = src_v[jj, sl] * 2 + cid
                    return c2

                lax.fori_loop(0, 8, mkidx, 0)
                return pltpu.async_copy(tab_h.at[idx2_v.at[b]],
                                        rows_v.at[b], sem_l)

            h_l = {0: mkidx_gather(0)}
            h_s = {}
            for jj in range(7):
                b = jj % 2
                if jj + 1 < 7:
                    if jj - 1 >= 0:
                        h_s[jj - 1].wait()
                    h_l[jj + 1] = mkidx_gather(jj + 1)
                h_l[jj].wait()

                def scale(g, c2, jj=jj, b=b):
                    exvec = ex_v[jj, pl.ds(g * 16, 16)]
                    for l in range(16):
                        a = exvec[l]
                        r = g * 16 + l
                        rows_v[b, r, pl.ds(0, 16)] = (
                            rows_v[b, r, pl.ds(0, 16)] * a)
                        rows_v[b, r, pl.ds(16, 16)] = (
                            rows_v[b, r, pl.ds(16, 16)] * a)
                    return c2

                lax.fori_loop(0, 8, scale, 0)
                h_s[jj] = pltpu.async_copy(rows_v.at[b],
                                           c_sh.at[dst_v.at[jj]], sem_s,
                                           add=True)
            h_s[5].wait()
            h_s[6].wait()
            return carry

        lax.fori_loop(0, _NJ5 // 7, stage, 0)
        plsc.subcore_barrier()
        pltpu.sync_copy(c_sh.at[pl.ds(tid * 3125, 3125)],
                        out_h.at[cid, pl.ds(tid * 3125, 3125)])

    return k(tab2, srcb, dstb, exb)


def _prep_gru(gp):
    w_ih, w_hh = gp['w_ih'], gp['w_hh']
    b_ih, b_hh = gp['b_ih'], gp['b_hh']
    out = {}
    for i, nm in enumerate(('r', 'z', 'n')):
        out['wi' + nm] = jnp.transpose(w_ih[i * 64:(i + 1) * 64])
        out['wh' + nm] = jnp.transpose(w_hh[i * 64:(i + 1) * 64])
        out['bi' + nm] = b_ih[i * 64:(i + 1) * 64].reshape(1, 64)
        out['bh' + nm] = b_hh[i * 64:(i + 1) * 64].reshape(1, 64)
    return out


def kernel(h, e, edge_index, graph_ids, params):
    p = params
    src = edge_index[0]
    dst = edge_index[1]

    pad = _EPAD - _E
    src_p = jnp.pad(src, (0, pad))
    dst_p = jnp.pad(dst, (0, pad))
    e_p = jnp.pad(e, ((0, pad), (0, 0)))
    valid = jnp.pad(jnp.ones((_E,), _F32), (0, pad))
    src3 = src_p.reshape(_NW, _NJ, 128)
    dst3 = dst_p.reshape(_NW, _NJ, 128)
    srcb = src_p.reshape(_NS, _NJ5, 128)
    dstb = dst_p.reshape(_NS, _NJ5, 128)
    arangeb = jnp.arange(_EPAD, dtype=jnp.int32).reshape(_NS, _NJ5, 128)
    valid3 = valid.reshape(_NW, _NJ, 128)
    gids3 = graph_ids.reshape(_NGB, _NB, 1)

    w1 = p['proj_edge1']['w']
    w2 = p['proj_edge2']['w']
    wpn0 = p['proj_node']['w']
    bpn0 = p['proj_node']['b'].reshape(1, 64)
    w1a, w1b = w1[:133], w1[133:]
    b1 = p['proj_edge1']['b'].reshape(1, 64)
    w2a, w2b = w2[:64], w2[64:]
    b2 = p['proj_edge2']['b'].reshape(1, 1)
    wt = p['edge_transform']['w']
    bt = p['edge_transform']['b'].reshape(1, 64)
    gru0 = _prep_gru(p['gru0'])

    hv_new, hs, ld, mxld = _tc_pre0(h, wpn0, bpn0, w1a, w2a, b2)
    hs_src = _sc_gather_rows(hs, src3)
    t, le2, mxle = _tc_edge0(hs_src, e_p, w1b, b1, wt, bt, w2b)
    m16 = jnp.broadcast_to(_leaky(mxld + mxle).reshape(1), (16,))
    ex3, s_parts = _sc_le_denom(le2.reshape(_NW, _NJ, 128), valid3,
                                dst3, ld.reshape(_N), m16)
    inv_s = _tc_inv_s(s_parts)
    cp = _sc_gather_scale_scatter(t.reshape(2 * _EPAD, 32), arangeb, dstb,
                                  ex3.reshape(_NS, _NJ5, 128))
    node = _tc_gru(cp[0], cp[1], inv_s.reshape(_N, 1), hv_new, gru0,
                   _N, _NB)

    for lp in p['gnn']:
        wpe = lp['proj_edge']['w']
        hp, ldst, lsrc, mxd, mxs = _tc_layer_pre(
            node, lp['proj_node']['w'], lp['proj_node']['b'].reshape(1, 64),
            wpe[:64], wpe[64:], lp['proj_edge']['b'].reshape(1, 1))
        m16 = jnp.broadcast_to(_leaky(mxd + mxs).reshape(1), (16,))
        ex3, s_parts = _sc_logits_denom(ldst.reshape(_N), lsrc.reshape(_N),
                                        dst3, src3, valid3, m16)
        inv_s = _tc_inv_s(s_parts)
        cp = _sc_gather_scale_scatter(hp.reshape(2 * _N, 32), srcb, dstb,
                                      ex3.reshape(_NS, _NJ5, 128))
        node = _tc_gru(cp[0], cp[1], inv_s.reshape(_N, 1), node,
                       _prep_gru(lp['gru']), _N, _NB)

    g_feats = _tc_readout_sum(node, gids3)
    for rp in p['readout']:
        wcl = rp['compute_logits']['w']
        z3, m = _tc_r2a(node, gids3, g_feats, wcl[:64], wcl[64:],
                        rp['compute_logits']['b'].reshape(1, 1))
        s, U = _tc_r2b(node, gids3, z3, m, rp['project_nodes']['w'],
                       rp['project_nodes']['b'].reshape(1, 64))
        g_feats = _tc_r2c(U, s, g_feats, _prep_gru(rp['gru']))
    return g_feats


# speedup vs baseline: 1.2006x; 1.0294x over previous
import functools

import jax
import jax.numpy as jnp
from jax import lax
from jax.experimental import pallas as pl
from jax.experimental.pallas import tpu as pltpu
from jax.experimental.pallas import tpu_sc as plsc

_N = 50000
_E = 800000
_H = 64
_G = 128
_NC = 2
_NS = 16
_NW = _NC * _NS
_EPAD = 802816
_CPW = _EPAD // _NW
_NJ = _CPW // 128
_CPT = _EPAD // _NS
_NJ5 = _CPT // 128
_NH = _N // 2
_CSH = 25600
_TRASH = 25300
_NB = 400
_NGB = _N // _NB
_EB = 512
_EGB = _EPAD // _EB

_F32 = jnp.float32


@functools.lru_cache(maxsize=1)
def _mesh():
    return plsc.VectorSubcoreMesh(core_axis_name="c", subcore_axis_name="s",
                                  num_cores=_NC, num_subcores=_NS)


def _leaky(x):
    return jnp.where(x >= 0, x, 0.01 * x)


def _elu(x):
    return jnp.where(x > 0, x, jnp.exp(x) - 1.0)


def _full(shape):
    return pl.BlockSpec(shape, lambda i: (0,) * len(shape))


def _tc_pre0(h, wpn, bpn, w1a, w2a, b2):
    def body(h_ref, wpn_ref, bpn_ref, w1a_ref, w2a_ref, b2_ref,
             hv_ref, hs_ref, ld_ref, md_ref):
        i = pl.program_id(0)
        hb = h_ref[...]
        hv = _leaky(jnp.dot(hb, wpn_ref[...], preferred_element_type=_F32)
                    + bpn_ref[...])
        hv_ref[...] = hv
        hs_ref[...] = jnp.dot(hb, w1a_ref[...], preferred_element_type=_F32)
        ld = jnp.dot(hv, w2a_ref[...], preferred_element_type=_F32) + b2_ref[...]
        ld_ref[...] = ld

        @pl.when(i == 0)
        def _():
            md_ref[...] = jnp.full((1, 1), -3e38, _F32)

        md_ref[...] = jnp.maximum(md_ref[...], jnp.max(ld)[None, None])

    return pl.pallas_call(
        body,
        grid=(_NGB,),
        in_specs=[pl.BlockSpec((_NB, 133), lambda i: (i, 0)),
                  _full((133, 64)), _full((1, 64)), _full((133, 64)),
                  _full((64, 1)), _full((1, 1))],
        out_specs=[pl.BlockSpec((_NB, 64), lambda i: (i, 0)),
                   pl.BlockSpec((_NB, 64), lambda i: (i, 0)),
                   pl.BlockSpec((_NB, 1), lambda i: (i, 0)),
                   _full((1, 1))],
        out_shape=[jax.ShapeDtypeStruct((_N, 64), _F32),
                   jax.ShapeDtypeStruct((_N, 64), _F32),
                   jax.ShapeDtypeStruct((_N, 1), _F32),
                   jax.ShapeDtypeStruct((1, 1), _F32)],
    )(h, wpn, bpn, w1a, w2a, b2)


def _tc_edge0(hs_src, e_p, w1b, b1, wt, bt, w2b):
    def body(hs_ref, e_ref, w1b_ref, b1_ref, wt_ref, bt_ref,
             w2b_ref, t_ref, le_ref, gm_ref):
        i = pl.program_id(0)
        he1 = _leaky(hs_ref[...]
                     + jnp.dot(e_ref[...], w1b_ref[...],
                               preferred_element_type=_F32) + b1_ref[...])
        t_ref[...] = jnp.dot(he1, wt_ref[...],
                             preferred_element_type=_F32) + bt_ref[...]
        le = jnp.dot(he1, w2b_ref[...], preferred_element_type=_F32)
        le_ref[...] = le

        @pl.when(i == 0)
        def _():
            gm_ref[...] = jnp.full((1, 1), -3e38, _F32)

        gm_ref[...] = jnp.maximum(gm_ref[...], jnp.max(le)[None, None])

    return pl.pallas_call(
        body,
        grid=(_EGB,),
        in_specs=[pl.BlockSpec((_EB, 64), lambda i: (i, 0)),
                  pl.BlockSpec((_EB, 14), lambda i: (i, 0)),
                  _full((14, 64)), _full((1, 64)), _full((64, 64)),
                  _full((1, 64)), _full((64, 1))],
        out_specs=[pl.BlockSpec((_EB, 64), lambda i: (i, 0)),
                   pl.BlockSpec((_EB, 1), lambda i: (i, 0)),
                   _full((1, 1))],
        out_shape=[jax.ShapeDtypeStruct((_EPAD, 64), _F32),
                   jax.ShapeDtypeStruct((_EPAD, 1), _F32),
                   jax.ShapeDtypeStruct((1, 1), _F32)],
    )(hs_src, e_p, w1b, b1, wt, bt, w2b)


def _tc_inv_s(s_parts):
    def body(sp_ref, out_ref):
        s = jnp.sum(sp_ref[...], axis=0, keepdims=True)
        out_ref[...] = 1.0 / jnp.where(s == 0.0, 1.0, s)

    return pl.pallas_call(
        body,
        grid=(1,),
        in_specs=[_full((_NW, _N))],
        out_specs=_full((1, _N)),
        out_shape=jax.ShapeDtypeStruct((1, _N), _F32),
    )(s_parts)


def _tc_gru(cpa, cpb, inv_n1, hprev, gw, n_rows, blk):
    def body(xa_ref, xb_ref, inv_ref, h_ref, wir, wiz, win, whr, whz, whn,
             bir, biz, bin_, bhr, bhz, bhn, out_ref):
        x = _elu(jnp.concatenate([xa_ref[...], xb_ref[...]], axis=1)
                 * inv_ref[...])
        hp = h_ref[...]
        dot = lambda a, b: jnp.dot(a, b[...], preferred_element_type=_F32)
        r = jax.nn.sigmoid(dot(x, wir) + bir[...] + dot(hp, whr) + bhr[...])
        z = jax.nn.sigmoid(dot(x, wiz) + biz[...] + dot(hp, whz) + bhz[...])
        n = jnp.tanh(dot(x, win) + bin_[...] + r * (dot(hp, whn) + bhn[...]))
        out_ref[...] = jax.nn.relu((1.0 - z) * n + z * hp)

    mats = [gw[k] for k in ('wir', 'wiz', 'win', 'whr', 'whz', 'whn')]
    vecs = [gw[k] for k in ('bir', 'biz', 'bin', 'bhr', 'bhz', 'bhn')]
    return pl.pallas_call(
        body,
        grid=(n_rows // blk,),
        in_specs=[pl.BlockSpec((blk, 32), lambda i: (i, 0)),
                  pl.BlockSpec((blk, 32), lambda i: (i, 0)),
                  pl.BlockSpec((blk, 1), lambda i: (i, 0)),
                  pl.BlockSpec((blk, 64), lambda i: (i, 0))]
                 + [_full((64, 64))] * 6 + [_full((1, 64))] * 6,
        out_specs=pl.BlockSpec((blk, 64), lambda i: (i, 0)),
        out_shape=jax.ShapeDtypeStruct((n_rows, 64), _F32),
    )(cpa, cpb, inv_n1, hprev, *mats, *vecs)


def _tc_layer_pre(node, wp, bp, wdst, wsrc, be):
    def body(n_ref, wp_ref, bp_ref, wd_ref, ws_ref, be_ref,
             hp_ref, ld_ref, ls_ref, md_ref, ms_ref):
        i = pl.program_id(0)
        nb = n_ref[...]
        hp_ref[...] = jnp.dot(nb, wp_ref[...],
                              preferred_element_type=_F32) + bp_ref[...]
        ldb = jnp.dot(nb, wd_ref[...], preferred_element_type=_F32) + be_ref[...]
        lsb = jnp.dot(nb, ws_ref[...], preferred_element_type=_F32)
        ld_ref[...] = ldb
        ls_ref[...] = lsb

        @pl.when(i == 0)
        def _():
            md_ref[...] = jnp.full((1, 1), -3e38, _F32)
            ms_ref[...] = jnp.full((1, 1), -3e38, _F32)

        md_ref[...] = jnp.maximum(md_ref[...], jnp.max(ldb)[None, None])
        ms_ref[...] = jnp.maximum(ms_ref[...], jnp.max(lsb)[None, None])

    return pl.pallas_call(
        body,
        grid=(_NGB,),
        in_specs=[pl.BlockSpec((_NB, 64), lambda i: (i, 0)),
                  _full((64, 64)), _full((1, 64)), _full((64, 1)),
                  _full((64, 1)), _full((1, 1))],
        out_specs=[pl.BlockSpec((_NB, 64), lambda i: (i, 0)),
                   pl.BlockSpec((_NB, 1), lambda i: (i, 0)),
                   pl.BlockSpec((_NB, 1), lambda i: (i, 0)),
                   _full((1, 1)), _full((1, 1))],
        out_shape=[jax.ShapeDtypeStruct((_N, 64), _F32),
                   jax.ShapeDtypeStruct((_N, 1), _F32),
                   jax.ShapeDtypeStruct((_N, 1), _F32),
                   jax.ShapeDtypeStruct((1, 1), _F32),
                   jax.ShapeDtypeStruct((1, 1), _F32)],
    )(node, wp, bp, wdst, wsrc, be)


def _onehot(gid_blk):
    iota = lax.broadcasted_iota(jnp.int32, (1, _G), 1)
    return jnp.where(gid_blk == iota, 1.0, 0.0).astype(_F32)


def _tc_readout_sum(node, gids3):
    def body(n_ref, g_ref, out_ref):
        i = pl.program_id(0)

        @pl.when(i == 0)
        def _():
            out_ref[...] = jnp.zeros((_G, 64), _F32)

        oh = _onehot(g_ref[0])
        hf = jax.nn.relu(n_ref[...])
        out_ref[...] += lax.dot_general(oh, hf, (((0,), (0,)), ((), ())),
                                        preferred_element_type=_F32)

    return pl.pallas_call(
        body,
        grid=(_NGB,),
        in_specs=[pl.BlockSpec((_NB, 64), lambda i: (i, 0)),
                  pl.BlockSpec((1, _NB, 1), lambda i: (i, 0, 0))],
        out_specs=_full((_G, 64)),
        out_shape=jax.ShapeDtypeStruct((_G, 64), _F32),
    )(node, gids3)


def _tc_r2a(node, gids3, g_feats, wclg, wclh, bz):
    def body(n_ref, g_ref, gf_ref, wg_ref, wh_ref, bz_ref, z_ref, m_ref):
        i = pl.program_id(0)

        @pl.when(i == 0)
        def _():
            m_ref[...] = jnp.full((1, _G), -3e38, _F32)

        zg = jnp.dot(jax.nn.relu(gf_ref[...]), wg_ref[...],
                     preferred_element_type=_F32)
        oh = _onehot(g_ref[0])
        zgn = jnp.dot(oh, zg, preferred_element_type=_F32)
        zh = jnp.dot(jax.nn.relu(n_ref[...]), wh_ref[...],
                     preferred_element_type=_F32)
        z = _leaky(zgn + zh + bz_ref[...])
        z_ref[0] = z
        cand = jnp.where(oh > 0, z, -3e38)
        m_ref[...] = jnp.maximum(m_ref[...],
                                 jnp.max(cand, axis=0, keepdims=True))

    return pl.pallas_call(
        body,
        grid=(_NGB,),
        in_specs=[pl.BlockSpec((_NB, 64), lambda i: (i, 0)),
                  pl.BlockSpec((1, _NB, 1), lambda i: (i, 0, 0)),
                  _full((_G, 64)), _full((64, 1)), _full((64, 1)),
                  _full((1, 1))],
        out_specs=[pl.BlockSpec((1, _NB, 1), lambda i: (i, 0, 0)),
                   _full((1, _G))],
        out_shape=[jax.ShapeDtypeStruct((_NGB, _NB, 1), _F32),
                   jax.ShapeDtypeStruct((1, _G), _F32)],
    )(node, gids3, g_feats, wclg, wclh, bz)


def _tc_r2b(node, gids3, z3, m, wpn, bpn):
    def body(n_ref, g_ref, z_ref, m_ref, wp_ref, bp_ref, s_ref, u_ref):
        i = pl.program_id(0)

        @pl.when(i == 0)
        def _():
            s_ref[...] = jnp.zeros((1, _G), _F32)
            u_ref[...] = jnp.zeros((_G, 64), _F32)

        oh = _onehot(g_ref[0])
        mn = lax.dot_general(oh, m_ref[...], (((1,), (1,)), ((), ())),
                             preferred_element_type=_F32)
        ex = jnp.exp(z_ref[0] - mn)
        s_ref[...] += lax.dot_general(ex, oh, (((0,), (0,)), ((), ())),
                                      preferred_element_type=_F32)
        hv2 = jnp.dot(jax.nn.relu(n_ref[...]), wp_ref[...],
                      preferred_element_type=_F32) + bp_ref[...]
        u_ref[...] += lax.dot_general(oh, hv2 * ex, (((0,), (0,)), ((), ())),
                                      preferred_element_type=_F32)

    return pl.pallas_call(
        body,
        grid=(_NGB,),
        in_specs=[pl.BlockSpec((_NB, 64), lambda i: (i, 0)),
                  pl.BlockSpec((1, _NB, 1), lambda i: (i, 0, 0)),
                  pl.BlockSpec((1, _NB, 1), lambda i: (i, 0, 0)),
                  _full((1, _G)), _full((64, 64)), _full((1, 64))],
        out_specs=[_full((1, _G)), _full((_G, 64))],
        out_shape=[jax.ShapeDtypeStruct((1, _G), _F32),
                   jax.ShapeDtypeStruct((_G, 64), _F32)],
    )(node, gids3, z3, m, wpn, bpn)


def _tc_r2c(U, s, g_feats, gw):
    def body(u_ref, s_ref, gf_ref, wir, wiz, win, whr, whz, whn,
             bir, biz, bin_, bhr, bhz, bhn, out_ref):
        sd = jnp.where(s_ref[...] == 0.0, 1.0, s_ref[...])
        g_repr = u_ref[...] * (1.0 / jnp.transpose(sd))
        x = _elu(g_repr)
        hp = gf_ref[...]
        dot = lambda a, b: jnp.dot(a, b[...], preferred_element_type=_F32)
        r = jax.nn.sigmoid(dot(x, wir) + bir[...] + dot(hp, whr) + bhr[...])
        z = jax.nn.sigmoid(dot(x, wiz) + biz[...] + dot(hp, whz) + bhz[...])
        n = jnp.tanh(dot(x, win) + bin_[...] + r * (dot(hp, whn) + bhn[...]))
        out_ref[...] = (1.0 - z) * n + z * hp

    mats = [gw[k] for k in ('wir', 'wiz', 'win', 'whr', 'whz', 'whn')]
    vecs = [gw[k] for k in ('bir', 'biz', 'bin', 'bhr', 'bhz', 'bhn')]
    return pl.pallas_call(
        body,
        grid=(1,),
        in_specs=[_full((_G, 64)), _full((1, _G)), _full((_G, 64))]
                 + [_full((64, 64))] * 6 + [_full((1, 64))] * 6,
        out_specs=_full((_G, 64)),
        out_shape=jax.ShapeDtypeStruct((_G, 64), _F32),
    )(U, s, g_feats, *mats, *vecs)


def _sc_gather_rows(table, idx3):
    @functools.partial(
        pl.kernel, mesh=_mesh(),
        compiler_params=pltpu.CompilerParams(needs_layout_passes=False, use_tc_tiling_on_sc=False),
        out_type=jax.ShapeDtypeStruct((_EPAD, 64), _F32),
        scratch_types=[pltpu.VMEM((_NJ, 128), jnp.int32),
                       pltpu.VMEM((7, 128, 64), _F32),
                       pltpu.SemaphoreType.DMA,
                       pltpu.SemaphoreType.DMA],
    )
    def k(table_h, idx_h, out_h, idx_v, rows_v, sem_g, sem_o):
        wid = lax.axis_index("s") * _NC + lax.axis_index("c")
        pltpu.sync_copy(idx_h.at[wid], idx_v)
        base = wid * _CPW
        nburst = 7

        def group(g, carry):
            j0 = g * nburst
            gh = [pltpu.async_copy(table_h.at[idx_v.at[j0 + b]],
                                   rows_v.at[b], sem_g)
                  for b in range(nburst)]
            oh = []
            for b in range(nburst):
                gh[b].wait()
                oh.append(pltpu.async_copy(
                    rows_v.at[b],
                    out_h.at[pl.ds((base + (j0 + b) * 128), 128)], sem_o))
            for b in range(nburst):
                oh[b].wait()
            return carry

        lax.fori_loop(0, _NJ // nburst, group, 0)

    return k(table, idx3)


def _sc_logits_denom(ldst, lsrc, dst3, src3, valid3, m16):
    st = 7

    @functools.partial(
        pl.kernel, mesh=_mesh(),
        compiler_params=pltpu.CompilerParams(needs_layout_passes=False, use_tc_tiling_on_sc=False),
        out_type=[jax.ShapeDtypeStruct((_NW, _NJ, 128), _F32),
                  jax.ShapeDtypeStruct((_NW, _N), _F32)],
        scratch_types=[pltpu.VMEM((_N,), _F32),
                       pltpu.VMEM((_N,), _F32),
                       pltpu.VMEM((_NH + 8,), _F32),
                       pltpu.VMEM((st, 128), jnp.int32),
                       pltpu.VMEM((st, 128), jnp.int32),
                       pltpu.VMEM((st, 128), _F32),
                       pltpu.VMEM((st, 128), _F32),
                       pltpu.VMEM((16,), _F32)],
    )
    def k(ld_h, ls_h, dst_h, src_h, val_h, m_h, ex_h, sp_h,
          ld_v, ls_v, sh_v, dst_v, src_v, val_v, ex_v, m_v):
        wid = lax.axis_index("s") * _NC + lax.axis_index("c")
        pltpu.sync_copy(ld_h, ld_v)
        pltpu.sync_copy(ls_h, ls_v)
        pltpu.sync_copy(m_h, m_v)
        mvec = m_v[...]
        zeros = jnp.zeros((16,), _F32)

        for p in (0, 1):
            lo = p * _NH

            def zbody(i, carry):
                sh_v[pl.ds(i * 16, 16)] = zeros
                return carry

            lax.fori_loop(0, (_NH + 8) // 16, zbody, 0)

            def outer(s, carry):
                pltpu.sync_copy(dst_h.at[wid, pl.ds(s * st, st)], dst_v)
                pltpu.sync_copy(src_h.at[wid, pl.ds(s * st, st)], src_v)
                if p == 0:
                    pltpu.sync_copy(val_h.at[wid, pl.ds(s * st, st)], val_v)

                def mid(j, c2):
                    def inner(v, c3):
                        sl = pl.ds(v * 16, 16)
                        dv = dst_v[j, sl]
                        lg = (plsc.load_gather(ld_v, [dv])
                              + plsc.load_gather(ls_v, [src_v[j, sl]]))
                        lg = jnp.where(lg >= 0, lg, 0.01 * lg)
                        ex = jnp.exp(lg - mvec)
                        if p == 0:
                            ex = ex * val_v[j, sl]
                            ex_v[j, sl] = ex
                        tgt = dv - lo
                        inr = jnp.logical_and(tgt >= 0, tgt < _NH)
                        plsc.addupdate_scatter(
                            sh_v, [jnp.where(inr, tgt, _NH)], ex, mask=inr)
                        return c3
                    return lax.fori_loop(0, 8, inner, c2)

                lax.fori_loop(0, st, mid, 0)
                if p == 0:
                    pltpu.sync_copy(ex_v, ex_h.at[wid, pl.ds(s * st, st)])
                return carry

            lax.fori_loop(0, _NJ // st, outer, 0)
            pltpu.sync_copy(sh_v.at[pl.ds(0, _NH)],
                            sp_h.at[wid, pl.ds(lo, _NH)])

    return k(ldst, lsrc, dst3, src3, valid3, m16)


def _sc_le_denom(le3, valid3, dst3, ldtab, m16):
    st = 7

    @functools.partial(
        pl.kernel, mesh=_mesh(),
        compiler_params=pltpu.CompilerParams(needs_layout_passes=False, use_tc_tiling_on_sc=False),
        out_type=[jax.ShapeDtypeStruct((_NW, _NJ, 128), _F32),
                  jax.ShapeDtypeStruct((_NW, _N), _F32)],
        scratch_types=[pltpu.VMEM((_N,), _F32),
                       pltpu.VMEM((_N,), _F32),
                       pltpu.VMEM((st, 128), _F32),
                       pltpu.VMEM((st, 128), _F32),
                       pltpu.VMEM((st, 128), jnp.int32),
                       pltpu.VMEM((st, 128), _F32),
                       pltpu.VMEM((16,), _F32)],
    )
    def k(le_h, val_h, dst_h, ld_h, m_h, ex_h, sp_h,
          ld_v, s_v, le_v, val_v, dst_v, ex_v, m_v):
        wid = lax.axis_index("s") * _NC + lax.axis_index("c")
        pltpu.sync_copy(ld_h, ld_v)
        pltpu.sync_copy(m_h, m_v)
        mvec = m_v[...]
        zeros = jnp.zeros((16,), _F32)

        def zbody(i, carry):
            s_v[pl.ds(i * 16, 16)] = zeros
            return carry

        lax.fori_loop(0, _N // 16, zbody, 0)

        def outer(s, carry):
            sl7 = pl.ds(s * st, st)
            pltpu.sync_copy(le_h.at[wid, sl7], le_v)
            pltpu.sync_copy(val_h.at[wid, sl7], val_v)
            pltpu.sync_copy(dst_h.at[wid, sl7], dst_v)

            def mid(j, c2):
                def inner(v, c3):
                    sl = pl.ds(v * 16, 16)
                    dv = dst_v[j, sl]
                    lg = plsc.load_gather(ld_v, [dv]) + le_v[j, sl]
                    lg = jnp.where(lg >= 0, lg, 0.01 * lg)
                    ex = jnp.exp(lg - mvec) * val_v[j, sl]
                    ex_v[j, sl] = ex
                    plsc.addupdate_scatter(s_v, [dv], ex)
                    return c3
                return lax.fori_loop(0, 8, inner, c2)

            lax.fori_loop(0, st, mid, 0)
            pltpu.sync_copy(ex_v, ex_h.at[wid, sl7])
            return carry

        lax.fori_loop(0, _NJ // st, outer, 0)
        pltpu.sync_copy(s_v, sp_h.at[wid])

    return k(le3, valid3, dst3, ldtab, m16)


def _sc_gather_scale_scatter(tab2, srcb, dstb, exb):
    _CS2 = 50048

    @functools.partial(
        pl.kernel, mesh=_mesh(),
        compiler_params=pltpu.CompilerParams(needs_layout_passes=False, use_tc_tiling_on_sc=False),
        out_type=jax.ShapeDtypeStruct((2, _N, 32), _F32),
        scratch_types=[pltpu.VMEM((7, 128), jnp.int32),
                       pltpu.VMEM((7, 128), jnp.int32),
                       pltpu.VMEM((7, 128), _F32),
                       pltpu.VMEM((3, 128, 32), _F32),
                       pltpu.VMEM((3, 128), jnp.int32),
                       pltpu.VMEM((128, 32), _F32),
                       pltpu.SemaphoreType.DMA,
                       pltpu.SemaphoreType.DMA,
                       pltpu.VMEM_SHARED((_CS2, 32), _F32)],
    )
    def k(tab_h, src_h, dst_h, ex_h, out_h, src_v, dst_v, ex_v,
          rows_v, idx2_v, zeros_v, sem_l, sem_s, c_sh):
        cid = lax.axis_index("c")
        tid = lax.axis_index("s")
        zeros = jnp.zeros((16,), _F32)

        def zb(i, carry):
            r = i // 2
            kk = i % 2
            zeros_v[r, pl.ds(kk * 16, 16)] = zeros
            return carry

        lax.fori_loop(0, 256, zb, 0)

        def zspmem(z, carry):
            pltpu.sync_copy(zeros_v,
                            c_sh.at[pl.ds(tid * 3128 + z * 128, 128)])
            return carry

        lax.fori_loop(0, 24, zspmem, 0)
        pltpu.sync_copy(zeros_v.at[pl.ds(0, 56)],
                        c_sh.at[pl.ds(tid * 3128 + 3072, 56)])
        plsc.subcore_barrier()

        def stage(s, carry):
            sl7 = pl.ds(s * 7, 7)
            pltpu.sync_copy(src_h.at[tid, sl7], src_v)
            pltpu.sync_copy(dst_h.at[tid, sl7], dst_v)
            pltpu.sync_copy(ex_h.at[tid, sl7], ex_v)
            def mkidx_gather(jj):
                b = jj % 3

                def mkidx(v, c2, jj=jj, b=b):
                    sl = pl.ds(v * 16, 16)
                    idx2_v[b, sl] = src_v[jj, sl] * 2 + cid
                    return c2

                lax.fori_loop(0, 8, mkidx, 0)
                return pltpu.async_copy(tab_h.at[idx2_v.at[b]],
                                        rows_v.at[b], sem_l)

            h_l = {0: mkidx_gather(0), 1: mkidx_gather(1)}
            h_s = {}
            for jj in range(7):
                b = jj % 3
                if jj + 2 < 7:
                    if jj - 1 >= 0:
                        h_s[jj - 1].wait()
                    h_l[jj + 2] = mkidx_gather(jj + 2)
                h_l[jj].wait()

                def scale(g, c2, jj=jj, b=b):
                    exvec = ex_v[jj, pl.ds(g * 16, 16)]
                    for l in range(16):
                        a = exvec[l]
                        r = g * 16 + l
                        rows_v[b, r, pl.ds(0, 16)] = (
                            rows_v[b, r, pl.ds(0, 16)] * a)
                        rows_v[b, r, pl.ds(16, 16)] = (
                            rows_v[b, r, pl.ds(16, 16)] * a)
                    return c2

                lax.fori_loop(0, 8, scale, 0)
                h_s[jj] = pltpu.async_copy(rows_v.at[b],
                                           c_sh.at[dst_v.at[jj]], sem_s,
                                           add=True)
            h_s[4].wait()
            h_s[5].wait()
            h_s[6].wait()
            return carry

        lax.fori_loop(0, _NJ5 // 7, stage, 0)
        plsc.subcore_barrier()
        pltpu.sync_copy(c_sh.at[pl.ds(tid * 3125, 3125)],
                        out_h.at[cid, pl.ds(tid * 3125, 3125)])

    return k(tab2, srcb, dstb, exb)


def _prep_gru(gp):
    w_ih, w_hh = gp['w_ih'], gp['w_hh']
    b_ih, b_hh = gp['b_ih'], gp['b_hh']
    out = {}
    for i, nm in enumerate(('r', 'z', 'n')):
        out['wi' + nm] = jnp.transpose(w_ih[i * 64:(i + 1) * 64])
        out['wh' + nm] = jnp.transpose(w_hh[i * 64:(i + 1) * 64])
        out['bi' + nm] = b_ih[i * 64:(i + 1) * 64].reshape(1, 64)
        out['bh' + nm] = b_hh[i * 64:(i + 1) * 64].reshape(1, 64)
    return out


def kernel(h, e, edge_index, graph_ids, params):
    p = params
    src = edge_index[0]
    dst = edge_index[1]

    pad = _EPAD - _E
    src_p = jnp.pad(src, (0, pad))
    dst_p = jnp.pad(dst, (0, pad))
    e_p = jnp.pad(e, ((0, pad), (0, 0)))
    valid = jnp.pad(jnp.ones((_E,), _F32), (0, pad))
    src3 = src_p.reshape(_NW, _NJ, 128)
    dst3 = dst_p.reshape(_NW, _NJ, 128)
    srcb = src_p.reshape(_NS, _NJ5, 128)
    dstb = dst_p.reshape(_NS, _NJ5, 128)
    arangeb = jnp.arange(_EPAD, dtype=jnp.int32).reshape(_NS, _NJ5, 128)
    valid3 = valid.reshape(_NW, _NJ, 128)
    gids3 = graph_ids.reshape(_NGB, _NB, 1)

    w1 = p['proj_edge1']['w']
    w2 = p['proj_edge2']['w']
    wpn0 = p['proj_node']['w']
    bpn0 = p['proj_node']['b'].reshape(1, 64)
    w1a, w1b = w1[:133], w1[133:]
    b1 = p['proj_edge1']['b'].reshape(1, 64)
    w2a, w2b = w2[:64], w2[64:]
    b2 = p['proj_edge2']['b'].reshape(1, 1)
    wt = p['edge_transform']['w']
    bt = p['edge_transform']['b'].reshape(1, 64)
    gru0 = _prep_gru(p['gru0'])

    hv_new, hs, ld, mxld = _tc_pre0(h, wpn0, bpn0, w1a, w2a, b2)
    hs_src = _sc_gather_rows(hs, src3)
    t, le2, mxle = _tc_edge0(hs_src, e_p, w1b, b1, wt, bt, w2b)
    m16 = jnp.broadcast_to(_leaky(mxld + mxle).reshape(1), (16,))
    ex3, s_parts = _sc_le_denom(le2.reshape(_NW, _NJ, 128), valid3,
                                dst3, ld.reshape(_N), m16)
    inv_s = _tc_inv_s(s_parts)
    cp = _sc_gather_scale_scatter(t.reshape(2 * _EPAD, 32), arangeb, dstb,
                                  ex3.reshape(_NS, _NJ5, 128))
    node = _tc_gru(cp[0], cp[1], inv_s.reshape(_N, 1), hv_new, gru0,
                   _N, _NB)

    for lp in p['gnn']:
        wpe = lp['proj_edge']['w']
        hp, ldst, lsrc, mxd, mxs = _tc_layer_pre(
            node, lp['proj_node']['w'], lp['proj_node']['b'].reshape(1, 64),
            wpe[:64], wpe[64:], lp['proj_edge']['b'].reshape(1, 1))
        m16 = jnp.broadcast_to(_leaky(mxd + mxs).reshape(1), (16,))
        ex3, s_parts = _sc_logits_denom(ldst.reshape(_N), lsrc.reshape(_N),
                                        dst3, src3, valid3, m16)
        inv_s = _tc_inv_s(s_parts)
        cp = _sc_gather_scale_scatter(hp.reshape(2 * _N, 32), srcb, dstb,
                                      ex3.reshape(_NS, _NJ5, 128))
        node = _tc_gru(cp[0], cp[1], inv_s.reshape(_N, 1), node,
                       _prep_gru(lp['gru']), _N, _NB)

    g_feats = _tc_readout_sum(node, gids3)
    for rp in p['readout']:
        wcl = rp['compute_logits']['w']
        z3, m = _tc_r2a(node, gids3, g_feats, wcl[:64], wcl[64:],
                        rp['compute_logits']['b'].reshape(1, 1))
        s, U = _tc_r2b(node, gids3, z3, m, rp['project_nodes']['w'],
                       rp['project_nodes']['b'].reshape(1, 64))
        g_feats = _tc_r2c(U, s, g_feats, _prep_gru(rp['gru']))
    return g_feats


# speedup vs baseline: 1.2288x; 1.0235x over previous
import functools

import jax
import jax.numpy as jnp
from jax import lax
from jax.experimental import pallas as pl
from jax.experimental.pallas import tpu as pltpu
from jax.experimental.pallas import tpu_sc as plsc

_N = 50000
_E = 800000
_H = 64
_G = 128
_NC = 2
_NS = 16
_NW = _NC * _NS
_EPAD = 802816
_CPW = _EPAD // _NW
_NJ = _CPW // 128
_CPT = _EPAD // _NS
_NJ5 = _CPT // 128
_NH = _N // 2
_CSH = 25600
_TRASH = 25300
_NB = 400
_NGB = _N // _NB
_EB = 512
_EGB = _EPAD // _EB

_F32 = jnp.float32


@functools.lru_cache(maxsize=1)
def _mesh():
    return plsc.VectorSubcoreMesh(core_axis_name="c", subcore_axis_name="s",
                                  num_cores=_NC, num_subcores=_NS)


def _leaky(x):
    return jnp.where(x >= 0, x, 0.01 * x)


def _elu(x):
    return jnp.where(x > 0, x, jnp.exp(x) - 1.0)


def _full(shape):
    return pl.BlockSpec(shape, lambda i: (0,) * len(shape))


def _tc_pre0(h, wpn, bpn, w1a, w2a, b2):
    def body(h_ref, wpn_ref, bpn_ref, w1a_ref, w2a_ref, b2_ref,
             hv_ref, hs_ref, ld_ref, md_ref):
        i = pl.program_id(0)
        hb = h_ref[...]
        hv = _leaky(jnp.dot(hb, wpn_ref[...], preferred_element_type=_F32)
                    + bpn_ref[...])
        hv_ref[...] = hv
        hs_ref[...] = jnp.dot(hb, w1a_ref[...], preferred_element_type=_F32)
        ld = jnp.dot(hv, w2a_ref[...], preferred_element_type=_F32) + b2_ref[...]
        ld_ref[...] = ld

        @pl.when(i == 0)
        def _():
            md_ref[...] = jnp.full((1, 1), -3e38, _F32)

        md_ref[...] = jnp.maximum(md_ref[...], jnp.max(ld)[None, None])

    return pl.pallas_call(
        body,
        grid=(_NGB,),
        in_specs=[pl.BlockSpec((_NB, 133), lambda i: (i, 0)),
                  _full((133, 64)), _full((1, 64)), _full((133, 64)),
                  _full((64, 1)), _full((1, 1))],
        out_specs=[pl.BlockSpec((_NB, 64), lambda i: (i, 0)),
                   pl.BlockSpec((_NB, 64), lambda i: (i, 0)),
                   pl.BlockSpec((_NB, 1), lambda i: (i, 0)),
                   _full((1, 1))],
        out_shape=[jax.ShapeDtypeStruct((_N, 64), _F32),
                   jax.ShapeDtypeStruct((_N, 64), _F32),
                   jax.ShapeDtypeStruct((_N, 1), _F32),
                   jax.ShapeDtypeStruct((1, 1), _F32)],
    )(h, wpn, bpn, w1a, w2a, b2)


def _tc_edge0(hs_src, e_p, w1b, b1, wt, bt, w2b):
    def body(hs_ref, e_ref, w1b_ref, b1_ref, wt_ref, bt_ref,
             w2b_ref, t_ref, le_ref, gm_ref):
        i = pl.program_id(0)
        he1 = _leaky(hs_ref[...]
                     + jnp.dot(e_ref[...], w1b_ref[...],
                               preferred_element_type=_F32) + b1_ref[...])
        t_ref[...] = jnp.dot(he1, wt_ref[...],
                             preferred_element_type=_F32) + bt_ref[...]
        le = jnp.dot(he1, w2b_ref[...], preferred_element_type=_F32)
        le_ref[...] = le

        @pl.when(i == 0)
        def _():
            gm_ref[...] = jnp.full((1, 1), -3e38, _F32)

        gm_ref[...] = jnp.maximum(gm_ref[...], jnp.max(le)[None, None])

    return pl.pallas_call(
        body,
        grid=(_EGB,),
        in_specs=[pl.BlockSpec((_EB, 64), lambda i: (i, 0)),
                  pl.BlockSpec((_EB, 14), lambda i: (i, 0)),
                  _full((14, 64)), _full((1, 64)), _full((64, 64)),
                  _full((1, 64)), _full((64, 1))],
        out_specs=[pl.BlockSpec((_EB, 64), lambda i: (i, 0)),
                   pl.BlockSpec((_EB, 1), lambda i: (i, 0)),
                   _full((1, 1))],
        out_shape=[jax.ShapeDtypeStruct((_EPAD, 64), _F32),
                   jax.ShapeDtypeStruct((_EPAD, 1), _F32),
                   jax.ShapeDtypeStruct((1, 1), _F32)],
    )(hs_src, e_p, w1b, b1, wt, bt, w2b)


def _tc_inv_s(s_parts):
    def body(sp_ref, out_ref):
        s = jnp.sum(sp_ref[...], axis=0, keepdims=True)
        out_ref[...] = 1.0 / jnp.where(s == 0.0, 1.0, s)

    return pl.pallas_call(
        body,
        grid=(1,),
        in_specs=[_full((_NW, _N))],
        out_specs=_full((1, _N)),
        out_shape=jax.ShapeDtypeStruct((1, _N), _F32),
    )(s_parts)


def _tc_gru(cpa, cpb, inv_n1, hprev, gw, n_rows, blk):
    def body(xa_ref, xb_ref, inv_ref, h_ref, wir, wiz, win, whr, whz, whn,
             bir, biz, bin_, bhr, bhz, bhn, out_ref):
        x = _elu(jnp.concatenate([xa_ref[...], xb_ref[...]], axis=1)
                 * inv_ref[...])
        hp = h_ref[...]
        dot = lambda a, b: jnp.dot(a, b[...], preferred_element_type=_F32)
        r = jax.nn.sigmoid(dot(x, wir) + bir[...] + dot(hp, whr) + bhr[...])
        z = jax.nn.sigmoid(dot(x, wiz) + biz[...] + dot(hp, whz) + bhz[...])
        n = jnp.tanh(dot(x, win) + bin_[...] + r * (dot(hp, whn) + bhn[...]))
        out_ref[...] = jax.nn.relu((1.0 - z) * n + z * hp)

    mats = [gw[k] for k in ('wir', 'wiz', 'win', 'whr', 'whz', 'whn')]
    vecs = [gw[k] for k in ('bir', 'biz', 'bin', 'bhr', 'bhz', 'bhn')]
    return pl.pallas_call(
        body,
        grid=(n_rows // blk,),
        in_specs=[pl.BlockSpec((blk, 32), lambda i: (i, 0)),
                  pl.BlockSpec((blk, 32), lambda i: (i, 0)),
                  pl.BlockSpec((blk, 1), lambda i: (i, 0)),
                  pl.BlockSpec((blk, 64), lambda i: (i, 0))]
                 + [_full((64, 64))] * 6 + [_full((1, 64))] * 6,
        out_specs=pl.BlockSpec((blk, 64), lambda i: (i, 0)),
        out_shape=jax.ShapeDtypeStruct((n_rows, 64), _F32),
    )(cpa, cpb, inv_n1, hprev, *mats, *vecs)


def _tc_gru_pre(cpa, cpb, inv_n1, hprev, gw, wp, bp, wdst, wsrc, be):
    def body(xa_ref, xb_ref, inv_ref, h_ref, wir, wiz, win, whr, whz, whn,
             bir, biz, bin_, bhr, bhz, bhn, wp_ref, bp_ref, wd_ref, ws_ref,
             be_ref, out_ref, hp_ref, ld_ref, ls_ref, md_ref, ms_ref):
        i = pl.program_id(0)
        x = _elu(jnp.concatenate([xa_ref[...], xb_ref[...]], axis=1)
                 * inv_ref[...])
        hp = h_ref[...]
        dot = lambda a, b: jnp.dot(a, b[...], preferred_element_type=_F32)
        r = jax.nn.sigmoid(dot(x, wir) + bir[...] + dot(hp, whr) + bhr[...])
        z = jax.nn.sigmoid(dot(x, wiz) + biz[...] + dot(hp, whz) + bhz[...])
        n = jnp.tanh(dot(x, win) + bin_[...] + r * (dot(hp, whn) + bhn[...]))
        nb = jax.nn.relu((1.0 - z) * n + z * hp)
        out_ref[...] = nb
        hp_ref[...] = dot(nb, wp_ref) + bp_ref[...]
        ldb = dot(nb, wd_ref) + be_ref[...]
        lsb = dot(nb, ws_ref)
        ld_ref[...] = ldb
        ls_ref[...] = lsb

        @pl.when(i == 0)
        def _():
            md_ref[...] = jnp.full((1, 1), -3e38, _F32)
            ms_ref[...] = jnp.full((1, 1), -3e38, _F32)

        md_ref[...] = jnp.maximum(md_ref[...], jnp.max(ldb)[None, None])
        ms_ref[...] = jnp.maximum(ms_ref[...], jnp.max(lsb)[None, None])

    mats = [gw[k] for k in ('wir', 'wiz', 'win', 'whr', 'whz', 'whn')]
    vecs = [gw[k] for k in ('bir', 'biz', 'bin', 'bhr', 'bhz', 'bhn')]
    return pl.pallas_call(
        body,
        grid=(_NGB,),
        in_specs=[pl.BlockSpec((_NB, 32), lambda i: (i, 0)),
                  pl.BlockSpec((_NB, 32), lambda i: (i, 0)),
                  pl.BlockSpec((_NB, 1), lambda i: (i, 0)),
                  pl.BlockSpec((_NB, 64), lambda i: (i, 0))]
                 + [_full((64, 64))] * 6 + [_full((1, 64))] * 6
                 + [_full((64, 64)), _full((1, 64)), _full((64, 1)),
                    _full((64, 1)), _full((1, 1))],
        out_specs=[pl.BlockSpec((_NB, 64), lambda i: (i, 0)),
                   pl.BlockSpec((_NB, 64), lambda i: (i, 0)),
                   pl.BlockSpec((_NB, 1), lambda i: (i, 0)),
                   pl.BlockSpec((_NB, 1), lambda i: (i, 0)),
                   _full((1, 1)), _full((1, 1))],
        out_shape=[jax.ShapeDtypeStruct((_N, 64), _F32),
                   jax.ShapeDtypeStruct((_N, 64), _F32),
                   jax.ShapeDtypeStruct((_N, 1), _F32),
                   jax.ShapeDtypeStruct((_N, 1), _F32),
                   jax.ShapeDtypeStruct((1, 1), _F32),
                   jax.ShapeDtypeStruct((1, 1), _F32)],
    )(cpa, cpb, inv_n1, hprev, *mats, *vecs, wp, bp, wdst, wsrc, be)


def _tc_layer_pre(node, wp, bp, wdst, wsrc, be):
    def body(n_ref, wp_ref, bp_ref, wd_ref, ws_ref, be_ref,
             hp_ref, ld_ref, ls_ref, md_ref, ms_ref):
        i = pl.program_id(0)
        nb = n_ref[...]
        hp_ref[...] = jnp.dot(nb, wp_ref[...],
                              preferred_element_type=_F32) + bp_ref[...]
        ldb = jnp.dot(nb, wd_ref[...], preferred_element_type=_F32) + be_ref[...]
        lsb = jnp.dot(nb, ws_ref[...], preferred_element_type=_F32)
        ld_ref[...] = ldb
        ls_ref[...] = lsb

        @pl.when(i == 0)
        def _():
            md_ref[...] = jnp.full((1, 1), -3e38, _F32)
            ms_ref[...] = jnp.full((1, 1), -3e38, _F32)

        md_ref[...] = jnp.maximum(md_ref[...], jnp.max(ldb)[None, None])
        ms_ref[...] = jnp.maximum(ms_ref[...], jnp.max(lsb)[None, None])

    return pl.pallas_call(
        body,
        grid=(_NGB,),
        in_specs=[pl.BlockSpec((_NB, 64), lambda i: (i, 0)),
                  _full((64, 64)), _full((1, 64)), _full((64, 1)),
                  _full((64, 1)), _full((1, 1))],
        out_specs=[pl.BlockSpec((_NB, 64), lambda i: (i, 0)),
                   pl.BlockSpec((_NB, 1), lambda i: (i, 0)),
                   pl.BlockSpec((_NB, 1), lambda i: (i, 0)),
                   _full((1, 1)), _full((1, 1))],
        out_shape=[jax.ShapeDtypeStruct((_N, 64), _F32),
                   jax.ShapeDtypeStruct((_N, 1), _F32),
                   jax.ShapeDtypeStruct((_N, 1), _F32),
                   jax.ShapeDtypeStruct((1, 1), _F32),
                   jax.ShapeDtypeStruct((1, 1), _F32)],
    )(node, wp, bp, wdst, wsrc, be)


def _onehot(gid_blk):
    iota = lax.broadcasted_iota(jnp.int32, (1, _G), 1)
    return jnp.where(gid_blk == iota, 1.0, 0.0).astype(_F32)


def _tc_readout_sum(node, gids3):
    def body(n_ref, g_ref, out_ref):
        i = pl.program_id(0)

        @pl.when(i == 0)
        def _():
            out_ref[...] = jnp.zeros((_G, 64), _F32)

        oh = _onehot(g_ref[0])
        hf = jax.nn.relu(n_ref[...])
        out_ref[...] += lax.dot_general(oh, hf, (((0,), (0,)), ((), ())),
                                        preferred_element_type=_F32)

    return pl.pallas_call(
        body,
        grid=(_NGB,),
        in_specs=[pl.BlockSpec((_NB, 64), lambda i: (i, 0)),
                  pl.BlockSpec((1, _NB, 1), lambda i: (i, 0, 0))],
        out_specs=_full((_G, 64)),
        out_shape=jax.ShapeDtypeStruct((_G, 64), _F32),
    )(node, gids3)


def _tc_r2a(node, gids3, g_feats, wclg, wclh, bz):
    def body(n_ref, g_ref, gf_ref, wg_ref, wh_ref, bz_ref, z_ref, m_ref):
        i = pl.program_id(0)

        @pl.when(i == 0)
        def _():
            m_ref[...] = jnp.full((1, _G), -3e38, _F32)

        zg = jnp.dot(jax.nn.relu(gf_ref[...]), wg_ref[...],
                     preferred_element_type=_F32)
        oh = _onehot(g_ref[0])
        zgn = jnp.dot(oh, zg, preferred_element_type=_F32)
        zh = jnp.dot(jax.nn.relu(n_ref[...]), wh_ref[...],
                     preferred_element_type=_F32)
        z = _leaky(zgn + zh + bz_ref[...])
        z_ref[0] = z
        cand = jnp.where(oh > 0, z, -3e38)
        m_ref[...] = jnp.maximum(m_ref[...],
                                 jnp.max(cand, axis=0, keepdims=True))

    return pl.pallas_call(
        body,
        grid=(_NGB,),
        in_specs=[pl.BlockSpec((_NB, 64), lambda i: (i, 0)),
                  pl.BlockSpec((1, _NB, 1), lambda i: (i, 0, 0)),
                  _full((_G, 64)), _full((64, 1)), _full((64, 1)),
                  _full((1, 1))],
        out_specs=[pl.BlockSpec((1, _NB, 1), lambda i: (i, 0, 0)),
                   _full((1, _G))],
        out_shape=[jax.ShapeDtypeStruct((_NGB, _NB, 1), _F32),
                   jax.ShapeDtypeStruct((1, _G), _F32)],
    )(node, gids3, g_feats, wclg, wclh, bz)


def _tc_r2b(node, gids3, z3, m, wpn, bpn):
    def body(n_ref, g_ref, z_ref, m_ref, wp_ref, bp_ref, s_ref, u_ref):
        i = pl.program_id(0)

        @pl.when(i == 0)
        def _():
            s_ref[...] = jnp.zeros((1, _G), _F32)
            u_ref[...] = jnp.zeros((_G, 64), _F32)

        oh = _onehot(g_ref[0])
        mn = lax.dot_general(oh, m_ref[...], (((1,), (1,)), ((), ())),
                             preferred_element_type=_F32)
        ex = jnp.exp(z_ref[0] - mn)
        s_ref[...] += lax.dot_general(ex, oh, (((0,), (0,)), ((), ())),
                                      preferred_element_type=_F32)
        hv2 = jnp.dot(jax.nn.relu(n_ref[...]), wp_ref[...],
                      preferred_element_type=_F32) + bp_ref[...]
        u_ref[...] += lax.dot_general(oh, hv2 * ex, (((0,), (0,)), ((), ())),
                                      preferred_element_type=_F32)

    return pl.pallas_call(
        body,
        grid=(_NGB,),
        in_specs=[pl.BlockSpec((_NB, 64), lambda i: (i, 0)),
                  pl.BlockSpec((1, _NB, 1), lambda i: (i, 0, 0)),
                  pl.BlockSpec((1, _NB, 1), lambda i: (i, 0, 0)),
                  _full((1, _G)), _full((64, 64)), _full((1, 64))],
        out_specs=[_full((1, _G)), _full((_G, 64))],
        out_shape=[jax.ShapeDtypeStruct((1, _G), _F32),
                   jax.ShapeDtypeStruct((_G, 64), _F32)],
    )(node, gids3, z3, m, wpn, bpn)


def _tc_r2c(U, s, g_feats, gw):
    def body(u_ref, s_ref, gf_ref, wir, wiz, win, whr, whz, whn,
             bir, biz, bin_, bhr, bhz, bhn, out_ref):
        sd = jnp.where(s_ref[...] == 0.0, 1.0, s_ref[...])
        g_repr = u_ref[...] * (1.0 / jnp.transpose(sd))
        x = _elu(g_repr)
        hp = gf_ref[...]
        dot = lambda a, b: jnp.dot(a, b[...], preferred_element_type=_F32)
        r = jax.nn.sigmoid(dot(x, wir) + bir[...] + dot(hp, whr) + bhr[...])
        z = jax.nn.sigmoid(dot(x, wiz) + biz[...] + dot(hp, whz) + bhz[...])
        n = jnp.tanh(dot(x, win) + bin_[...] + r * (dot(hp, whn) + bhn[...]))
        out_ref[...] = (1.0 - z) * n + z * hp

    mats = [gw[k] for k in ('wir', 'wiz', 'win', 'whr', 'whz', 'whn')]
    vecs = [gw[k] for k in ('bir', 'biz', 'bin', 'bhr', 'bhz', 'bhn')]
    return pl.pallas_call(
        body,
        grid=(1,),
        in_specs=[_full((_G, 64)), _full((1, _G)), _full((_G, 64))]
                 + [_full((64, 64))] * 6 + [_full((1, 64))] * 6,
        out_specs=_full((_G, 64)),
        out_shape=jax.ShapeDtypeStruct((_G, 64), _F32),
    )(U, s, g_feats, *mats, *vecs)


def _sc_gather_rows(table, idx3):
    @functools.partial(
        pl.kernel, mesh=_mesh(),
        compiler_params=pltpu.CompilerParams(needs_layout_passes=False, use_tc_tiling_on_sc=False),
        out_type=jax.ShapeDtypeStruct((_EPAD, 64), _F32),
        scratch_types=[pltpu.VMEM((_NJ, 128), jnp.int32),
                       pltpu.VMEM((7, 128, 64), _F32),
                       pltpu.SemaphoreType.DMA,
                       pltpu.SemaphoreType.DMA],
    )
    def k(table_h, idx_h, out_h, idx_v, rows_v, sem_g, sem_o):
        wid = lax.axis_index("s") * _NC + lax.axis_index("c")
        pltpu.sync_copy(idx_h.at[wid], idx_v)
        base = wid * _CPW
        nburst = 7

        def group(g, carry):
            j0 = g * nburst
            gh = [pltpu.async_copy(table_h.at[idx_v.at[j0 + b]],
                                   rows_v.at[b], sem_g)
                  for b in range(nburst)]
            oh = []
            for b in range(nburst):
                gh[b].wait()
                oh.append(pltpu.async_copy(
                    rows_v.at[b],
                    out_h.at[pl.ds((base + (j0 + b) * 128), 128)], sem_o))
            for b in range(nburst):
                oh[b].wait()
            return carry

        lax.fori_loop(0, _NJ // nburst, group, 0)

    return k(table, idx3)


def _sc_logits_denom(ldst, lsrc, dst3, src3, valid3, m16):
    st = 7

    @functools.partial(
        pl.kernel, mesh=_mesh(),
        compiler_params=pltpu.CompilerParams(needs_layout_passes=False, use_tc_tiling_on_sc=False),
        out_type=[jax.ShapeDtypeStruct((_NW, _NJ, 128), _F32),
                  jax.ShapeDtypeStruct((_NW, _N), _F32)],
        scratch_types=[pltpu.VMEM((_N,), _F32),
                       pltpu.VMEM((_N,), _F32),
                       pltpu.VMEM((_NH + 8,), _F32),
                       pltpu.VMEM((st, 128), jnp.int32),
                       pltpu.VMEM((st, 128), jnp.int32),
                       pltpu.VMEM((st, 128), _F32),
                       pltpu.VMEM((st, 128), _F32),
                       pltpu.VMEM((16,), _F32)],
    )
    def k(ld_h, ls_h, dst_h, src_h, val_h, m_h, ex_h, sp_h,
          ld_v, ls_v, sh_v, dst_v, src_v, val_v, ex_v, m_v):
        wid = lax.axis_index("s") * _NC + lax.axis_index("c")
        pltpu.sync_copy(ld_h, ld_v)
        pltpu.sync_copy(ls_h, ls_v)
        pltpu.sync_copy(m_h, m_v)
        mvec = m_v[...]
        zeros = jnp.zeros((16,), _F32)

        for p in (0, 1):
            lo = p * _NH

            def zbody(i, carry):
                sh_v[pl.ds(i * 16, 16)] = zeros
                return carry

            lax.fori_loop(0, (_NH + 8) // 16, zbody, 0)

            def outer(s, carry):
                pltpu.sync_copy(dst_h.at[wid, pl.ds(s * st, st)], dst_v)
                pltpu.sync_copy(src_h.at[wid, pl.ds(s * st, st)], src_v)
                if p == 0:
                    pltpu.sync_copy(val_h.at[wid, pl.ds(s * st, st)], val_v)

                def mid(j, c2):
                    def inner(v, c3):
                        sl = pl.ds(v * 16, 16)
                        dv = dst_v[j, sl]
                        lg = (plsc.load_gather(ld_v, [dv])
                              + plsc.load_gather(ls_v, [src_v[j, sl]]))
                        lg = jnp.where(lg >= 0, lg, 0.01 * lg)
                        ex = jnp.exp(lg - mvec)
                        if p == 0:
                            ex = ex * val_v[j, sl]
                            ex_v[j, sl] = ex
                        tgt = dv - lo
                        inr = jnp.logical_and(tgt >= 0, tgt < _NH)
                        plsc.addupdate_scatter(
                            sh_v, [jnp.where(inr, tgt, _NH)], ex, mask=inr)
                        return c3
                    return lax.fori_loop(0, 8, inner, c2)

                lax.fori_loop(0, st, mid, 0)
                if p == 0:
                    pltpu.sync_copy(ex_v, ex_h.at[wid, pl.ds(s * st, st)])
                return carry

            lax.fori_loop(0, _NJ // st, outer, 0)
            pltpu.sync_copy(sh_v.at[pl.ds(0, _NH)],
                            sp_h.at[wid, pl.ds(lo, _NH)])

    return k(ldst, lsrc, dst3, src3, valid3, m16)


def _sc_le_denom(le3, valid3, dst3, ldtab, m16):
    st = 7

    @functools.partial(
        pl.kernel, mesh=_mesh(),
        compiler_params=pltpu.CompilerParams(needs_layout_passes=False, use_tc_tiling_on_sc=False),
        out_type=[jax.ShapeDtypeStruct((_NW, _NJ, 128), _F32),
                  jax.ShapeDtypeStruct((_NW, _N), _F32)],
        scratch_types=[pltpu.VMEM((_N,), _F32),
                       pltpu.VMEM((_N,), _F32),
                       pltpu.VMEM((st, 128), _F32),
                       pltpu.VMEM((st, 128), _F32),
                       pltpu.VMEM((st, 128), jnp.int32),
                       pltpu.VMEM((st, 128), _F32),
                       pltpu.VMEM((16,), _F32)],
    )
    def k(le_h, val_h, dst_h, ld_h, m_h, ex_h, sp_h,
          ld_v, s_v, le_v, val_v, dst_v, ex_v, m_v):
        wid = lax.axis_index("s") * _NC + lax.axis_index("c")
        pltpu.sync_copy(ld_h, ld_v)
        pltpu.sync_copy(m_h, m_v)
        mvec = m_v[...]
        zeros = jnp.zeros((16,), _F32)

        def zbody(i, carry):
            s_v[pl.ds(i * 16, 16)] = zeros
            return carry

        lax.fori_loop(0, _N // 16, zbody, 0)

        def outer(s, carry):
            sl7 = pl.ds(s * st, st)
            pltpu.sync_copy(le_h.at[wid, sl7], le_v)
            pltpu.sync_copy(val_h.at[wid, sl7], val_v)
            pltpu.sync_copy(dst_h.at[wid, sl7], dst_v)

            def mid(j, c2):
                def inner(v, c3):
                    sl = pl.ds(v * 16, 16)
                    dv = dst_v[j, sl]
                    lg = plsc.load_gather(ld_v, [dv]) + le_v[j, sl]
                    lg = jnp.where(lg >= 0, lg, 0.01 * lg)
                    ex = jnp.exp(lg - mvec) * val_v[j, sl]
                    ex_v[j, sl] = ex
                    plsc.addupdate_scatter(s_v, [dv], ex)
                    return c3
                return lax.fori_loop(0, 8, inner, c2)

            lax.fori_loop(0, st, mid, 0)
            pltpu.sync_copy(ex_v, ex_h.at[wid, sl7])
            return carry

        lax.fori_loop(0, _NJ // st, outer, 0)
        pltpu.sync_copy(s_v, sp_h.at[wid])

    return k(le3, valid3, dst3, ldtab, m16)


def _sc_gather_scale_scatter(tab2, srcb, dstb, exb):
    _CS2 = 50048

    @functools.partial(
        pl.kernel, mesh=_mesh(),
        compiler_params=pltpu.CompilerParams(needs_layout_passes=False, use_tc_tiling_on_sc=False),
        out_type=jax.ShapeDtypeStruct((2, _N, 32), _F32),
        scratch_types=[pltpu.VMEM((7, 128), jnp.int32),
                       pltpu.VMEM((7, 128), jnp.int32),
                       pltpu.VMEM((7, 128), _F32),
                       pltpu.VMEM((3, 128, 32), _F32),
                       pltpu.VMEM((3, 128), jnp.int32),
                       pltpu.VMEM((128, 32), _F32),
                       pltpu.SemaphoreType.DMA,
                       pltpu.SemaphoreType.DMA,
                       pltpu.VMEM_SHARED((_CS2, 32), _F32)],
    )
    def k(tab_h, src_h, dst_h, ex_h, out_h, src_v, dst_v, ex_v,
          rows_v, idx2_v, zeros_v, sem_l, sem_s, c_sh):
        cid = lax.axis_index("c")
        tid = lax.axis_index("s")
        zeros = jnp.zeros((16,), _F32)

        def zb(i, carry):
            r = i // 2
            kk = i % 2
            zeros_v[r, pl.ds(kk * 16, 16)] = zeros
            return carry

        lax.fori_loop(0, 256, zb, 0)

        def zspmem(z, carry):
            pltpu.sync_copy(zeros_v,
                            c_sh.at[pl.ds(tid * 3128 + z * 128, 128)])
            return carry

        lax.fori_loop(0, 24, zspmem, 0)
        pltpu.sync_copy(zeros_v.at[pl.ds(0, 56)],
                        c_sh.at[pl.ds(tid * 3128 + 3072, 56)])
        plsc.subcore_barrier()

        def stage(s, carry):
            sl7 = pl.ds(s * 7, 7)
            pltpu.sync_copy(src_h.at[tid, sl7], src_v)
            pltpu.sync_copy(dst_h.at[tid, sl7], dst_v)
            pltpu.sync_copy(ex_h.at[tid, sl7], ex_v)
            def mkidx_gather(jj):
                b = jj % 3

                def mkidx(v, c2, jj=jj, b=b):
                    sl = pl.ds(v * 16, 16)
                    idx2_v[b, sl] = src_v[jj, sl] * 2 + cid
                    return c2

                lax.fori_loop(0, 8, mkidx, 0)
                return pltpu.async_copy(tab_h.at[idx2_v.at[b]],
                                        rows_v.at[b], sem_l)

            h_l = {0: mkidx_gather(0), 1: mkidx_gather(1)}
            h_s = {}
            for jj in range(7):
                b = jj % 3
                if jj + 2 < 7:
                    if jj - 1 >= 0:
                        h_s[jj - 1].wait()
                    h_l[jj + 2] = mkidx_gather(jj + 2)
                h_l[jj].wait()

                def scale(g, c2, jj=jj, b=b):
                    exvec = ex_v[jj, pl.ds(g * 16, 16)]
                    for l in range(16):
                        a = exvec[l]
                        r = g * 16 + l
                        rows_v[b, r, pl.ds(0, 16)] = (
                            rows_v[b, r, pl.ds(0, 16)] * a)
                        rows_v[b, r, pl.ds(16, 16)] = (
                            rows_v[b, r, pl.ds(16, 16)] * a)
                    return c2

                lax.fori_loop(0, 8, scale, 0)
                h_s[jj] = pltpu.async_copy(rows_v.at[b],
                                           c_sh.at[dst_v.at[jj]], sem_s,
                                           add=True)
            h_s[4].wait()
            h_s[5].wait()
            h_s[6].wait()
            return carry

        lax.fori_loop(0, _NJ5 // 7, stage, 0)
        plsc.subcore_barrier()
        pltpu.sync_copy(c_sh.at[pl.ds(tid * 3125, 3125)],
                        out_h.at[cid, pl.ds(tid * 3125, 3125)])

    return k(tab2, srcb, dstb, exb)


def _prep_gru(gp):
    w_ih, w_hh = gp['w_ih'], gp['w_hh']
    b_ih, b_hh = gp['b_ih'], gp['b_hh']
    out = {}
    for i, nm in enumerate(('r', 'z', 'n')):
        out['wi' + nm] = jnp.transpose(w_ih[i * 64:(i + 1) * 64])
        out['wh' + nm] = jnp.transpose(w_hh[i * 64:(i + 1) * 64])
        out['bi' + nm] = b_ih[i * 64:(i + 1) * 64].reshape(1, 64)
        out['bh' + nm] = b_hh[i * 64:(i + 1) * 64].reshape(1, 64)
    return out


def kernel(h, e, edge_index, graph_ids, params):
    p = params
    src = edge_index[0]
    dst = edge_index[1]

    pad = _EPAD - _E
    src_p = jnp.pad(src, (0, pad))
    dst_p = jnp.pad(dst, (0, pad))
    e_p = jnp.pad(e, ((0, pad), (0, 0)))
    valid = jnp.pad(jnp.ones((_E,), _F32), (0, pad))
    src3 = src_p.reshape(_NW, _NJ, 128)
    dst3 = dst_p.reshape(_NW, _NJ, 128)
    srcb = src_p.reshape(_NS, _NJ5, 128)
    dstb = dst_p.reshape(_NS, _NJ5, 128)
    arangeb = jnp.arange(_EPAD, dtype=jnp.int32).reshape(_NS, _NJ5, 128)
    valid3 = valid.reshape(_NW, _NJ, 128)
    gids3 = graph_ids.reshape(_NGB, _NB, 1)

    w1 = p['proj_edge1']['w']
    w2 = p['proj_edge2']['w']
    wpn0 = p['proj_node']['w']
    bpn0 = p['proj_node']['b'].reshape(1, 64)
    w1a, w1b = w1[:133], w1[133:]
    b1 = p['proj_edge1']['b'].reshape(1, 64)
    w2a, w2b = w2[:64], w2[64:]
    b2 = p['proj_edge2']['b'].reshape(1, 1)
    wt = p['edge_transform']['w']
    bt = p['edge_transform']['b'].reshape(1, 64)
    gru0 = _prep_gru(p['gru0'])

    hv_new, hs, ld, mxld = _tc_pre0(h, wpn0, bpn0, w1a, w2a, b2)
    hs_src = _sc_gather_rows(hs, src3)
    t, le2, mxle = _tc_edge0(hs_src, e_p, w1b, b1, wt, bt, w2b)
    m16 = jnp.broadcast_to(_leaky(mxld + mxle).reshape(1), (16,))
    ex3, s_parts = _sc_le_denom(le2.reshape(_NW, _NJ, 128), valid3,
                                dst3, ld.reshape(_N), m16)
    inv_s = _tc_inv_s(s_parts)
    cp = _sc_gather_scale_scatter(t.reshape(2 * _EPAD, 32), arangeb, dstb,
                                  ex3.reshape(_NS, _NJ5, 128))

    gnn = p['gnn']

    def pre_args(lp):
        wpe = lp['proj_edge']['w']
        return (lp['proj_node']['w'], lp['proj_node']['b'].reshape(1, 64),
                wpe[:64], wpe[64:], lp['proj_edge']['b'].reshape(1, 1))

    node, hp, ldst, lsrc, mxd, mxs = _tc_gru_pre(
        cp[0], cp[1], inv_s.reshape(_N, 1), hv_new, gru0, *pre_args(gnn[0]))
    for li, lp in enumerate(gnn):
        m16 = jnp.broadcast_to(_leaky(mxd + mxs).reshape(1), (16,))
        ex3, s_parts = _sc_logits_denom(ldst.reshape(_N), lsrc.reshape(_N),
                                        dst3, src3, valid3, m16)
        inv_s = _tc_inv_s(s_parts)
        cp = _sc_gather_scale_scatter(hp.reshape(2 * _N, 32), srcb, dstb,
                                      ex3.reshape(_NS, _NJ5, 128))
        if li + 1 < len(gnn):
            node, hp, ldst, lsrc, mxd, mxs = _tc_gru_pre(
                cp[0], cp[1], inv_s.reshape(_N, 1), node,
                _prep_gru(lp['gru']), *pre_args(gnn[li + 1]))
        else:
            node = _tc_gru(cp[0], cp[1], inv_s.reshape(_N, 1), node,
                           _prep_gru(lp['gru']), _N, _NB)

    g_feats = _tc_readout_sum(node, gids3)
    for rp in p['readout']:
        wcl = rp['compute_logits']['w']
        z3, m = _tc_r2a(node, gids3, g_feats, wcl[:64], wcl[64:],
                        rp['compute_logits']['b'].reshape(1, 1))
        s, U = _tc_r2b(node, gids3, z3, m, rp['project_nodes']['w'],
                       rp['project_nodes']['b'].reshape(1, 64))
        g_feats = _tc_r2c(U, s, g_feats, _prep_gru(rp['gru']))
    return g_feats


# speedup vs baseline: 1.2300x; 1.0010x over previous
import functools

import jax
import jax.numpy as jnp
from jax import lax
from jax.experimental import pallas as pl
from jax.experimental.pallas import tpu as pltpu
from jax.experimental.pallas import tpu_sc as plsc

_N = 50000
_E = 800000
_H = 64
_G = 128
_NC = 2
_NS = 16
_NW = _NC * _NS
_EPAD = 802816
_CPW = _EPAD // _NW
_NJ = _CPW // 128
_CPT = _EPAD // _NS
_NJ5 = _CPT // 128
_NH = _N // 2
_CSH = 25600
_TRASH = 25300
_NB = 400
_NGB = _N // _NB
_EB = 512
_EGB = _EPAD // _EB

_F32 = jnp.float32


@functools.lru_cache(maxsize=1)
def _mesh():
    return plsc.VectorSubcoreMesh(core_axis_name="c", subcore_axis_name="s",
                                  num_cores=_NC, num_subcores=_NS)


def _leaky(x):
    return jnp.where(x >= 0, x, 0.01 * x)


def _elu(x):
    return jnp.where(x > 0, x, jnp.exp(x) - 1.0)


def _full(shape):
    return pl.BlockSpec(shape, lambda i: (0,) * len(shape))


def _tc_pre0(h, wpn, bpn, w1a, w2a, b2):
    def body(h_ref, wpn_ref, bpn_ref, w1a_ref, w2a_ref, b2_ref,
             hv_ref, hs_ref, ld_ref, md_ref):
        i = pl.program_id(0)
        hb = h_ref[...]
        hv = _leaky(jnp.dot(hb, wpn_ref[...], preferred_element_type=_F32)
                    + bpn_ref[...])
        hv_ref[...] = hv
        hs_ref[...] = jnp.dot(hb, w1a_ref[...], preferred_element_type=_F32)
        ld = jnp.dot(hv, w2a_ref[...], preferred_element_type=_F32) + b2_ref[...]
        ld_ref[...] = ld

        @pl.when(i == 0)
        def _():
            md_ref[...] = jnp.full((1, 1), -3e38, _F32)

        md_ref[...] = jnp.maximum(md_ref[...], jnp.max(ld)[None, None])

    return pl.pallas_call(
        body,
        grid=(_NGB,),
        in_specs=[pl.BlockSpec((_NB, 133), lambda i: (i, 0)),
                  _full((133, 64)), _full((1, 64)), _full((133, 64)),
                  _full((64, 1)), _full((1, 1))],
        out_specs=[pl.BlockSpec((_NB, 64), lambda i: (i, 0)),
                   pl.BlockSpec((_NB, 64), lambda i: (i, 0)),
                   pl.BlockSpec((_NB, 1), lambda i: (i, 0)),
                   _full((1, 1))],
        out_shape=[jax.ShapeDtypeStruct((_N, 64), _F32),
                   jax.ShapeDtypeStruct((_N, 64), _F32),
                   jax.ShapeDtypeStruct((_N, 1), _F32),
                   jax.ShapeDtypeStruct((1, 1), _F32)],
    )(h, wpn, bpn, w1a, w2a, b2)


def _tc_edge0(hs_src, e_p, w1b, b1, wt, bt, w2b):
    def body(hs_ref, e_ref, w1b_ref, b1_ref, wt_ref, bt_ref,
             w2b_ref, t_ref, le_ref, gm_ref):
        i = pl.program_id(0)
        he1 = _leaky(hs_ref[...]
                     + jnp.dot(e_ref[...], w1b_ref[...],
                               preferred_element_type=_F32) + b1_ref[...])
        t_ref[...] = jnp.dot(he1, wt_ref[...],
                             preferred_element_type=_F32) + bt_ref[...]
        le = jnp.dot(he1, w2b_ref[...], preferred_element_type=_F32)
        le_ref[...] = le

        @pl.when(i == 0)
        def _():
            gm_ref[...] = jnp.full((1, 1), -3e38, _F32)

        gm_ref[...] = jnp.maximum(gm_ref[...], jnp.max(le)[None, None])

    return pl.pallas_call(
        body,
        grid=(_EGB,),
        in_specs=[pl.BlockSpec((_EB, 64), lambda i: (i, 0)),
                  pl.BlockSpec((_EB, 14), lambda i: (i, 0)),
                  _full((14, 64)), _full((1, 64)), _full((64, 64)),
                  _full((1, 64)), _full((64, 1))],
        out_specs=[pl.BlockSpec((_EB, 64), lambda i: (i, 0)),
                   pl.BlockSpec((_EB, 1), lambda i: (i, 0)),
                   _full((1, 1))],
        out_shape=[jax.ShapeDtypeStruct((_EPAD, 64), _F32),
                   jax.ShapeDtypeStruct((_EPAD, 1), _F32),
                   jax.ShapeDtypeStruct((1, 1), _F32)],
    )(hs_src, e_p, w1b, b1, wt, bt, w2b)


def _tc_inv_s(s_parts):
    def body(sp_ref, out_ref):
        s = jnp.sum(sp_ref[...], axis=0, keepdims=True)
        out_ref[...] = 1.0 / jnp.where(s == 0.0, 1.0, s)

    return pl.pallas_call(
        body,
        grid=(1,),
        in_specs=[_full((_NW, _N))],
        out_specs=_full((1, _N)),
        out_shape=jax.ShapeDtypeStruct((1, _N), _F32),
    )(s_parts)


def _tc_gru(cpa, cpb, inv_n1, hprev, gw, n_rows, blk):
    def body(xa_ref, xb_ref, inv_ref, h_ref, wir, wiz, win, whr, whz, whn,
             bir, biz, bin_, bhr, bhz, bhn, out_ref):
        x = _elu(jnp.concatenate([xa_ref[...], xb_ref[...]], axis=1)
                 * inv_ref[...])
        hp = h_ref[...]
        dot = lambda a, b: jnp.dot(a, b[...], preferred_element_type=_F32)
        r = jax.nn.sigmoid(dot(x, wir) + bir[...] + dot(hp, whr) + bhr[...])
        z = jax.nn.sigmoid(dot(x, wiz) + biz[...] + dot(hp, whz) + bhz[...])
        n = jnp.tanh(dot(x, win) + bin_[...] + r * (dot(hp, whn) + bhn[...]))
        out_ref[...] = jax.nn.relu((1.0 - z) * n + z * hp)

    mats = [gw[k] for k in ('wir', 'wiz', 'win', 'whr', 'whz', 'whn')]
    vecs = [gw[k] for k in ('bir', 'biz', 'bin', 'bhr', 'bhz', 'bhn')]
    return pl.pallas_call(
        body,
        grid=(n_rows // blk,),
        in_specs=[pl.BlockSpec((blk, 32), lambda i: (i, 0)),
                  pl.BlockSpec((blk, 32), lambda i: (i, 0)),
                  pl.BlockSpec((blk, 1), lambda i: (i, 0)),
                  pl.BlockSpec((blk, 64), lambda i: (i, 0))]
                 + [_full((64, 64))] * 6 + [_full((1, 64))] * 6,
        out_specs=pl.BlockSpec((blk, 64), lambda i: (i, 0)),
        out_shape=jax.ShapeDtypeStruct((n_rows, 64), _F32),
    )(cpa, cpb, inv_n1, hprev, *mats, *vecs)


def _tc_gru_pre(cpa, cpb, inv_n1, hprev, gw, wp, bp, wdst, wsrc, be):
    def body(xa_ref, xb_ref, inv_ref, h_ref, wir, wiz, win, whr, whz, whn,
             bir, biz, bin_, bhr, bhz, bhn, wp_ref, bp_ref, wd_ref, ws_ref,
             be_ref, out_ref, hp_ref, ld_ref, ls_ref, md_ref, ms_ref):
        i = pl.program_id(0)
        x = _elu(jnp.concatenate([xa_ref[...], xb_ref[...]], axis=1)
                 * inv_ref[...])
        hp = h_ref[...]
        dot = lambda a, b: jnp.dot(a, b[...], preferred_element_type=_F32)
        r = jax.nn.sigmoid(dot(x, wir) + bir[...] + dot(hp, whr) + bhr[...])
        z = jax.nn.sigmoid(dot(x, wiz) + biz[...] + dot(hp, whz) + bhz[...])
        n = jnp.tanh(dot(x, win) + bin_[...] + r * (dot(hp, whn) + bhn[...]))
        nb = jax.nn.relu((1.0 - z) * n + z * hp)
        out_ref[...] = nb
        hp_ref[...] = dot(nb, wp_ref) + bp_ref[...]
        ldb = dot(nb, wd_ref) + be_ref[...]
        lsb = dot(nb, ws_ref)
        ld_ref[...] = ldb
        ls_ref[...] = lsb

        @pl.when(i == 0)
        def _():
            md_ref[...] = jnp.full((1, 1), -3e38, _F32)
            ms_ref[...] = jnp.full((1, 1), -3e38, _F32)

        md_ref[...] = jnp.maximum(md_ref[...], jnp.max(ldb)[None, None])
        ms_ref[...] = jnp.maximum(ms_ref[...], jnp.max(lsb)[None, None])

    mats = [gw[k] for k in ('wir', 'wiz', 'win', 'whr', 'whz', 'whn')]
    vecs = [gw[k] for k in ('bir', 'biz', 'bin', 'bhr', 'bhz', 'bhn')]
    return pl.pallas_call(
        body,
        grid=(_NGB,),
        in_specs=[pl.BlockSpec((_NB, 32), lambda i: (i, 0)),
                  pl.BlockSpec((_NB, 32), lambda i: (i, 0)),
                  pl.BlockSpec((_NB, 1), lambda i: (i, 0)),
                  pl.BlockSpec((_NB, 64), lambda i: (i, 0))]
                 + [_full((64, 64))] * 6 + [_full((1, 64))] * 6
                 + [_full((64, 64)), _full((1, 64)), _full((64, 1)),
                    _full((64, 1)), _full((1, 1))],
        out_specs=[pl.BlockSpec((_NB, 64), lambda i: (i, 0)),
                   pl.BlockSpec((_NB, 64), lambda i: (i, 0)),
                   pl.BlockSpec((_NB, 1), lambda i: (i, 0)),
                   pl.BlockSpec((_NB, 1), lambda i: (i, 0)),
                   _full((1, 1)), _full((1, 1))],
        out_shape=[jax.ShapeDtypeStruct((_N, 64), _F32),
                   jax.ShapeDtypeStruct((_N, 64), _F32),
                   jax.ShapeDtypeStruct((_N, 1), _F32),
                   jax.ShapeDtypeStruct((_N, 1), _F32),
                   jax.ShapeDtypeStruct((1, 1), _F32),
                   jax.ShapeDtypeStruct((1, 1), _F32)],
    )(cpa, cpb, inv_n1, hprev, *mats, *vecs, wp, bp, wdst, wsrc, be)


def _tc_layer_pre(node, wp, bp, wdst, wsrc, be):
    def body(n_ref, wp_ref, bp_ref, wd_ref, ws_ref, be_ref,
             hp_ref, ld_ref, ls_ref, md_ref, ms_ref):
        i = pl.program_id(0)
        nb = n_ref[...]
        hp_ref[...] = jnp.dot(nb, wp_ref[...],
                              preferred_element_type=_F32) + bp_ref[...]
        ldb = jnp.dot(nb, wd_ref[...], preferred_element_type=_F32) + be_ref[...]
        lsb = jnp.dot(nb, ws_ref[...], preferred_element_type=_F32)
        ld_ref[...] = ldb
        ls_ref[...] = lsb

        @pl.when(i == 0)
        def _():
            md_ref[...] = jnp.full((1, 1), -3e38, _F32)
            ms_ref[...] = jnp.full((1, 1), -3e38, _F32)

        md_ref[...] = jnp.maximum(md_ref[...], jnp.max(ldb)[None, None])
        ms_ref[...] = jnp.maximum(ms_ref[...], jnp.max(lsb)[None, None])

    return pl.pallas_call(
        body,
        grid=(_NGB,),
        in_specs=[pl.BlockSpec((_NB, 64), lambda i: (i, 0)),
                  _full((64, 64)), _full((1, 64)), _full((64, 1)),
                  _full((64, 1)), _full((1, 1))],
        out_specs=[pl.BlockSpec((_NB, 64), lambda i: (i, 0)),
                   pl.BlockSpec((_NB, 1), lambda i: (i, 0)),
                   pl.BlockSpec((_NB, 1), lambda i: (i, 0)),
                   _full((1, 1)), _full((1, 1))],
        out_shape=[jax.ShapeDtypeStruct((_N, 64), _F32),
                   jax.ShapeDtypeStruct((_N, 1), _F32),
                   jax.ShapeDtypeStruct((_N, 1), _F32),
                   jax.ShapeDtypeStruct((1, 1), _F32),
                   jax.ShapeDtypeStruct((1, 1), _F32)],
    )(node, wp, bp, wdst, wsrc, be)


def _onehot(gid_blk):
    iota = lax.broadcasted_iota(jnp.int32, (1, _G), 1)
    return jnp.where(gid_blk == iota, 1.0, 0.0).astype(_F32)


def _tc_readout_sum(node, gids3):
    def body(n_ref, g_ref, out_ref):
        i = pl.program_id(0)

        @pl.when(i == 0)
        def _():
            out_ref[...] = jnp.zeros((_G, 64), _F32)

        oh = _onehot(g_ref[0])
        hf = jax.nn.relu(n_ref[...])
        out_ref[...] += lax.dot_general(oh, hf, (((0,), (0,)), ((), ())),
                                        preferred_element_type=_F32)

    return pl.pallas_call(
        body,
        grid=(_NGB,),
        in_specs=[pl.BlockSpec((_NB, 64), lambda i: (i, 0)),
                  pl.BlockSpec((1, _NB, 1), lambda i: (i, 0, 0))],
        out_specs=_full((_G, 64)),
        out_shape=jax.ShapeDtypeStruct((_G, 64), _F32),
    )(node, gids3)


def _tc_r2a(node, gids3, g_feats, wclg, wclh, bz):
    def body(n_ref, g_ref, gf_ref, wg_ref, wh_ref, bz_ref, z_ref, m_ref):
        i = pl.program_id(0)

        @pl.when(i == 0)
        def _():
            m_ref[...] = jnp.full((1, _G), -3e38, _F32)

        zg = jnp.dot(jax.nn.relu(gf_ref[...]), wg_ref[...],
                     preferred_element_type=_F32)
        oh = _onehot(g_ref[0])
        zgn = jnp.dot(oh, zg, preferred_element_type=_F32)
        zh = jnp.dot(jax.nn.relu(n_ref[...]), wh_ref[...],
                     preferred_element_type=_F32)
        z = _leaky(zgn + zh + bz_ref[...])
        z_ref[0] = z
        cand = jnp.where(oh > 0, z, -3e38)
        m_ref[...] = jnp.maximum(m_ref[...],
                                 jnp.max(cand, axis=0, keepdims=True))

    return pl.pallas_call(
        body,
        grid=(_NGB,),
        in_specs=[pl.BlockSpec((_NB, 64), lambda i: (i, 0)),
                  pl.BlockSpec((1, _NB, 1), lambda i: (i, 0, 0)),
                  _full((_G, 64)), _full((64, 1)), _full((64, 1)),
                  _full((1, 1))],
        out_specs=[pl.BlockSpec((1, _NB, 1), lambda i: (i, 0, 0)),
                   _full((1, _G))],
        out_shape=[jax.ShapeDtypeStruct((_NGB, _NB, 1), _F32),
                   jax.ShapeDtypeStruct((1, _G), _F32)],
    )(node, gids3, g_feats, wclg, wclh, bz)


def _tc_r2bc(node, gids3, z3, m, wpn, bpn, g_feats, gw):
    def body(n_ref, g_ref, z_ref, m_ref, wp_ref, bp_ref, gf_ref,
             wir, wiz, win, whr, whz, whn,
             bir, biz, bin_, bhr, bhz, bhn, out_ref, s_ref, u_ref):
        i = pl.program_id(0)

        @pl.when(i == 0)
        def _():
            s_ref[...] = jnp.zeros((1, _G), _F32)
            u_ref[...] = jnp.zeros((_G, 64), _F32)

        oh = _onehot(g_ref[0])
        mn = lax.dot_general(oh, m_ref[...], (((1,), (1,)), ((), ())),
                             preferred_element_type=_F32)
        ex = jnp.exp(z_ref[0] - mn)
        s_ref[...] += lax.dot_general(ex, oh, (((0,), (0,)), ((), ())),
                                      preferred_element_type=_F32)
        hv2 = jnp.dot(jax.nn.relu(n_ref[...]), wp_ref[...],
                      preferred_element_type=_F32) + bp_ref[...]
        u_ref[...] += lax.dot_general(oh, hv2 * ex, (((0,), (0,)), ((), ())),
                                      preferred_element_type=_F32)

        @pl.when(i == _NGB - 1)
        def _():
            sd = jnp.where(s_ref[...] == 0.0, 1.0, s_ref[...])
            x = _elu(u_ref[...] * (1.0 / jnp.transpose(sd)))
            hp = gf_ref[...]
            dot = lambda a, b: jnp.dot(a, b[...],
                                       preferred_element_type=_F32)
            r = jax.nn.sigmoid(dot(x, wir) + bir[...]
                               + dot(hp, whr) + bhr[...])
            z = jax.nn.sigmoid(dot(x, wiz) + biz[...]
                               + dot(hp, whz) + bhz[...])
            n = jnp.tanh(dot(x, win) + bin_[...]
                         + r * (dot(hp, whn) + bhn[...]))
            out_ref[...] = (1.0 - z) * n + z * hp

    mats = [gw[k] for k in ('wir', 'wiz', 'win', 'whr', 'whz', 'whn')]
    vecs = [gw[k] for k in ('bir', 'biz', 'bin', 'bhr', 'bhz', 'bhn')]
    return pl.pallas_call(
        body,
        grid=(_NGB,),
        in_specs=[pl.BlockSpec((_NB, 64), lambda i: (i, 0)),
                  pl.BlockSpec((1, _NB, 1), lambda i: (i, 0, 0)),
                  pl.BlockSpec((1, _NB, 1), lambda i: (i, 0, 0)),
                  _full((1, _G)), _full((64, 64)), _full((1, 64)),
                  _full((_G, 64))]
                 + [_full((64, 64))] * 6 + [_full((1, 64))] * 6,
        out_specs=_full((_G, 64)),
        out_shape=jax.ShapeDtypeStruct((_G, 64), _F32),
        scratch_shapes=[pltpu.VMEM((1, _G), _F32),
                        pltpu.VMEM((_G, 64), _F32)],
    )(node, gids3, z3, m, wpn, bpn, g_feats, *mats, *vecs)


def _sc_gather_rows(table, idx3):
    @functools.partial(
        pl.kernel, mesh=_mesh(),
        compiler_params=pltpu.CompilerParams(needs_layout_passes=False, use_tc_tiling_on_sc=False),
        out_type=jax.ShapeDtypeStruct((_EPAD, 64), _F32),
        scratch_types=[pltpu.VMEM((_NJ, 128), jnp.int32),
                       pltpu.VMEM((7, 128, 64), _F32),
                       pltpu.SemaphoreType.DMA,
                       pltpu.SemaphoreType.DMA],
    )
    def k(table_h, idx_h, out_h, idx_v, rows_v, sem_g, sem_o):
        wid = lax.axis_index("s") * _NC + lax.axis_index("c")
        pltpu.sync_copy(idx_h.at[wid], idx_v)
        base = wid * _CPW
        nburst = 7

        def group(g, carry):
            j0 = g * nburst
            gh = [pltpu.async_copy(table_h.at[idx_v.at[j0 + b]],
                                   rows_v.at[b], sem_g)
                  for b in range(nburst)]
            oh = []
            for b in range(nburst):
                gh[b].wait()
                oh.append(pltpu.async_copy(
                    rows_v.at[b],
                    out_h.at[pl.ds((base + (j0 + b) * 128), 128)], sem_o))
            for b in range(nburst):
                oh[b].wait()
            return carry

        lax.fori_loop(0, _NJ // nburst, group, 0)

    return k(table, idx3)


def _sc_logits_denom(ldst, lsrc, dst3, src3, valid3, m16):
    st = 7

    @functools.partial(
        pl.kernel, mesh=_mesh(),
        compiler_params=pltpu.CompilerParams(needs_layout_passes=False, use_tc_tiling_on_sc=False),
        out_type=[jax.ShapeDtypeStruct((_NW, _NJ, 128), _F32),
                  jax.ShapeDtypeStruct((_NW, _N), _F32)],
        scratch_types=[pltpu.VMEM((_N,), _F32),
                       pltpu.VMEM((_N,), _F32),
                       pltpu.VMEM((_NH + 8,), _F32),
                       pltpu.VMEM((st, 128), jnp.int32),
                       pltpu.VMEM((st, 128), jnp.int32),
                       pltpu.VMEM((st, 128), _F32),
                       pltpu.VMEM((st, 128), _F32),
                       pltpu.VMEM((16,), _F32)],
    )
    def k(ld_h, ls_h, dst_h, src_h, val_h, m_h, ex_h, sp_h,
          ld_v, ls_v, sh_v, dst_v, src_v, val_v, ex_v, m_v):
        wid = lax.axis_index("s") * _NC + lax.axis_index("c")
        pltpu.sync_copy(ld_h, ld_v)
        pltpu.sync_copy(ls_h, ls_v)
        pltpu.sync_copy(m_h, m_v)
        mvec = m_v[...]
        zeros = jnp.zeros((16,), _F32)

        for p in (0, 1):
            lo = p * _NH

            def zbody(i, carry):
                sh_v[pl.ds(i * 16, 16)] = zeros
                return carry

            lax.fori_loop(0, (_NH + 8) // 16, zbody, 0)

            def outer(s, carry):
                pltpu.sync_copy(dst_h.at[wid, pl.ds(s * st, st)], dst_v)
                pltpu.sync_copy(src_h.at[wid, pl.ds(s * st, st)], src_v)
                if p == 0:
                    pltpu.sync_copy(val_h.at[wid, pl.ds(s * st, st)], val_v)

                def mid(j, c2):
                    def inner(v, c3):
                        sl = pl.ds(v * 16, 16)
                        dv = dst_v[j, sl]
                        lg = (plsc.load_gather(ld_v, [dv])
                              + plsc.load_gather(ls_v, [src_v[j, sl]]))
                        lg = jnp.where(lg >= 0, lg, 0.01 * lg)
                        ex = jnp.exp(lg - mvec)
                        if p == 0:
                            ex = ex * val_v[j, sl]
                            ex_v[j, sl] = ex
                        tgt = dv - lo
                        inr = jnp.logical_and(tgt >= 0, tgt < _NH)
                        plsc.addupdate_scatter(
                            sh_v, [jnp.where(inr, tgt, _NH)], ex, mask=inr)
                        return c3
                    return lax.fori_loop(0, 8, inner, c2)

                lax.fori_loop(0, st, mid, 0)
                if p == 0:
                    pltpu.sync_copy(ex_v, ex_h.at[wid, pl.ds(s * st, st)])
                return carry

            lax.fori_loop(0, _NJ // st, outer, 0)
            pltpu.sync_copy(sh_v.at[pl.ds(0, _NH)],
                            sp_h.at[wid, pl.ds(lo, _NH)])

    return k(ldst, lsrc, dst3, src3, valid3, m16)


def _sc_le_denom(le3, valid3, dst3, ldtab, m16):
    st = 7

    @functools.partial(
        pl.kernel, mesh=_mesh(),
        compiler_params=pltpu.CompilerParams(needs_layout_passes=False, use_tc_tiling_on_sc=False),
        out_type=[jax.ShapeDtypeStruct((_NW, _NJ, 128), _F32),
                  jax.ShapeDtypeStruct((_NW, _N), _F32)],
        scratch_types=[pltpu.VMEM((_N,), _F32),
                       pltpu.VMEM((_N,), _F32),
                       pltpu.VMEM((st, 128), _F32),
                       pltpu.VMEM((st, 128), _F32),
                       pltpu.VMEM((st, 128), jnp.int32),
                       pltpu.VMEM((st, 128), _F32),
                       pltpu.VMEM((16,), _F32)],
    )
    def k(le_h, val_h, dst_h, ld_h, m_h, ex_h, sp_h,
          ld_v, s_v, le_v, val_v, dst_v, ex_v, m_v):
        wid = lax.axis_index("s") * _NC + lax.axis_index("c")
        pltpu.sync_copy(ld_h, ld_v)
        pltpu.sync_copy(m_h, m_v)
        mvec = m_v[...]
        zeros = jnp.zeros((16,), _F32)

        def zbody(i, carry):
            s_v[pl.ds(i * 16, 16)] = zeros
            return carry

        lax.fori_loop(0, _N // 16, zbody, 0)

        def outer(s, carry):
            sl7 = pl.ds(s * st, st)
            pltpu.sync_copy(le_h.at[wid, sl7], le_v)
            pltpu.sync_copy(val_h.at[wid, sl7], val_v)
            pltpu.sync_copy(dst_h.at[wid, sl7], dst_v)

            def mid(j, c2):
                def inner(v, c3):
                    sl = pl.ds(v * 16, 16)
                    dv = dst_v[j, sl]
                    lg = plsc.load_gather(ld_v, [dv]) + le_v[j, sl]
                    lg = jnp.where(lg >= 0, lg, 0.01 * lg)
                    ex = jnp.exp(lg - mvec) * val_v[j, sl]
                    ex_v[j, sl] = ex
                    plsc.addupdate_scatter(s_v, [dv], ex)
                    return c3
                return lax.fori_loop(0, 8, inner, c2)

            lax.fori_loop(0, st, mid, 0)
            pltpu.sync_copy(ex_v, ex_h.at[wid, sl7])
            return carry

        lax.fori_loop(0, _NJ // st, outer, 0)
        pltpu.sync_copy(s_v, sp_h.at[wid])

    return k(le3, valid3, dst3, ldtab, m16)


def _sc_gather_scale_scatter(tab2, srcb, dstb, exb):
    _CS2 = 50048

    @functools.partial(
        pl.kernel, mesh=_mesh(),
        compiler_params=pltpu.CompilerParams(needs_layout_passes=False, use_tc_tiling_on_sc=False),
        out_type=jax.ShapeDtypeStruct((2, _N, 32), _F32),
        scratch_types=[pltpu.VMEM((7, 128), jnp.int32),
                       pltpu.VMEM((7, 128), jnp.int32),
                       pltpu.VMEM((7, 128), _F32),
                       pltpu.VMEM((3, 128, 32), _F32),
                       pltpu.VMEM((3, 128), jnp.int32),
                       pltpu.VMEM((128, 32), _F32),
                       pltpu.SemaphoreType.DMA,
                       pltpu.SemaphoreType.DMA,
                       pltpu.VMEM_SHARED((_CS2, 32), _F32)],
    )
    def k(tab_h, src_h, dst_h, ex_h, out_h, src_v, dst_v, ex_v,
          rows_v, idx2_v, zeros_v, sem_l, sem_s, c_sh):
        cid = lax.axis_index("c")
        tid = lax.axis_index("s")
        zeros = jnp.zeros((16,), _F32)

        def zb(i, carry):
            r = i // 2
            kk = i % 2
            zeros_v[r, pl.ds(kk * 16, 16)] = zeros
            return carry

        lax.fori_loop(0, 256, zb, 0)

        def zspmem(z, carry):
            pltpu.sync_copy(zeros_v,
                            c_sh.at[pl.ds(tid * 3128 + z * 128, 128)])
            return carry

        lax.fori_loop(0, 24, zspmem, 0)
        pltpu.sync_copy(zeros_v.at[pl.ds(0, 56)],
                        c_sh.at[pl.ds(tid * 3128 + 3072, 56)])
        plsc.subcore_barrier()

        def stage(s, carry):
            sl7 = pl.ds(s * 7, 7)
            pltpu.sync_copy(src_h.at[tid, sl7], src_v)
            pltpu.sync_copy(dst_h.at[tid, sl7], dst_v)
            pltpu.sync_copy(ex_h.at[tid, sl7], ex_v)
            def mkidx_gather(jj):
                b = jj % 3

                def mkidx(v, c2, jj=jj, b=b):
                    sl = pl.ds(v * 16, 16)
                    idx2_v[b, sl] = src_v[jj, sl] * 2 + cid
                    return c2

                lax.fori_loop(0, 8, mkidx, 0)
                return pltpu.async_copy(tab_h.at[idx2_v.at[b]],
                                        rows_v.at[b], sem_l)

            h_l = {0: mkidx_gather(0), 1: mkidx_gather(1)}
            h_s = {}
            for jj in range(7):
                b = jj % 3
                if jj + 2 < 7:
                    if jj - 1 >= 0:
                        h_s[jj - 1].wait()
                    h_l[jj + 2] = mkidx_gather(jj + 2)
                h_l[jj].wait()

                def scale(g, c2, jj=jj, b=b):
                    exvec = ex_v[jj, pl.ds(g * 16, 16)]
                    for l in range(16):
                        a = exvec[l]
                        r = g * 16 + l
                        rows_v[b, r, pl.ds(0, 16)] = (
                            rows_v[b, r, pl.ds(0, 16)] * a)
                        rows_v[b, r, pl.ds(16, 16)] = (
                            rows_v[b, r, pl.ds(16, 16)] * a)
                    return c2

                lax.fori_loop(0, 8, scale, 0)
                h_s[jj] = pltpu.async_copy(rows_v.at[b],
                                           c_sh.at[dst_v.at[jj]], sem_s,
                                           add=True)
            h_s[4].wait()
            h_s[5].wait()
            h_s[6].wait()
            return carry

        lax.fori_loop(0, _NJ5 // 7, stage, 0)
        plsc.subcore_barrier()
        pltpu.sync_copy(c_sh.at[pl.ds(tid * 3125, 3125)],
                        out_h.at[cid, pl.ds(tid * 3125, 3125)])

    return k(tab2, srcb, dstb, exb)


def _prep_gru(gp):
    w_ih, w_hh = gp['w_ih'], gp['w_hh']
    b_ih, b_hh = gp['b_ih'], gp['b_hh']
    out = {}
    for i, nm in enumerate(('r', 'z', 'n')):
        out['wi' + nm] = jnp.transpose(w_ih[i * 64:(i + 1) * 64])
        out['wh' + nm] = jnp.transpose(w_hh[i * 64:(i + 1) * 64])
        out['bi' + nm] = b_ih[i * 64:(i + 1) * 64].reshape(1, 64)
        out['bh' + nm] = b_hh[i * 64:(i + 1) * 64].reshape(1, 64)
    return out


def kernel(h, e, edge_index, graph_ids, params):
    p = params
    src = edge_index[0]
    dst = edge_index[1]

    pad = _EPAD - _E
    src_p = jnp.pad(src, (0, pad))
    dst_p = jnp.pad(dst, (0, pad))
    e_p = jnp.pad(e, ((0, pad), (0, 0)))
    valid = jnp.pad(jnp.ones((_E,), _F32), (0, pad))
    src3 = src_p.reshape(_NW, _NJ, 128)
    dst3 = dst_p.reshape(_NW, _NJ, 128)
    srcb = src_p.reshape(_NS, _NJ5, 128)
    dstb = dst_p.reshape(_NS, _NJ5, 128)
    arangeb = jnp.arange(_EPAD, dtype=jnp.int32).reshape(_NS, _NJ5, 128)
    valid3 = valid.reshape(_NW, _NJ, 128)
    gids3 = graph_ids.reshape(_NGB, _NB, 1)

    w1 = p['proj_edge1']['w']
    w2 = p['proj_edge2']['w']
    wpn0 = p['proj_node']['w']
    bpn0 = p['proj_node']['b'].reshape(1, 64)
    w1a, w1b = w1[:133], w1[133:]
    b1 = p['proj_edge1']['b'].reshape(1, 64)
    w2a, w2b = w2[:64], w2[64:]
    b2 = p['proj_edge2']['b'].reshape(1, 1)
    wt = p['edge_transform']['w']
    bt = p['edge_transform']['b'].reshape(1, 64)
    gru0 = _prep_gru(p['gru0'])

    hv_new, hs, ld, mxld = _tc_pre0(h, wpn0, bpn0, w1a, w2a, b2)
    hs_src = _sc_gather_rows(hs, src3)
    t, le2, mxle = _tc_edge0(hs_src, e_p, w1b, b1, wt, bt, w2b)
    m16 = jnp.broadcast_to(_leaky(mxld + mxle).reshape(1), (16,))
    ex3, s_parts = _sc_le_denom(le2.reshape(_NW, _NJ, 128), valid3,
                                dst3, ld.reshape(_N), m16)
    inv_s = _tc_inv_s(s_parts)
    cp = _sc_gather_scale_scatter(t.reshape(2 * _EPAD, 32), arangeb, dstb,
                                  ex3.reshape(_NS, _NJ5, 128))

    gnn = p['gnn']

    def pre_args(lp):
        wpe = lp['proj_edge']['w']
        return (lp['proj_node']['w'], lp['proj_node']['b'].reshape(1, 64),
                wpe[:64], wpe[64:], lp['proj_edge']['b'].reshape(1, 1))

    node, hp, ldst, lsrc, mxd, mxs = _tc_gru_pre(
        cp[0], cp[1], inv_s.reshape(_N, 1), hv_new, gru0, *pre_args(gnn[0]))
    for li, lp in enumerate(gnn):
        m16 = jnp.broadcast_to(_leaky(mxd + mxs).reshape(1), (16,))
        ex3, s_parts = _sc_logits_denom(ldst.reshape(_N), lsrc.reshape(_N),
                                        dst3, src3, valid3, m16)
        inv_s = _tc_inv_s(s_parts)
        cp = _sc_gather_scale_scatter(hp.reshape(2 * _N, 32), srcb, dstb,
                                      ex3.reshape(_NS, _NJ5, 128))
        if li + 1 < len(gnn):
            node, hp, ldst, lsrc, mxd, mxs = _tc_gru_pre(
                cp[0], cp[1], inv_s.reshape(_N, 1), node,
                _prep_gru(lp['gru']), *pre_args(gnn[li + 1]))
        else:
            node = _tc_gru(cp[0], cp[1], inv_s.reshape(_N, 1), node,
                           _prep_gru(lp['gru']), _N, _NB)

    g_feats = _tc_readout_sum(node, gids3)
    for rp in p['readout']:
        wcl = rp['compute_logits']['w']
        z3, m = _tc_r2a(node, gids3, g_feats, wcl[:64], wcl[64:],
                        rp['compute_logits']['b'].reshape(1, 1))
        g_feats = _tc_r2bc(node, gids3, z3, m, rp['project_nodes']['w'],
                           rp['project_nodes']['b'].reshape(1, 64),
                           g_feats, _prep_gru(rp['gru']))
    return g_feats
